# Initial kernel scaffold; baseline (speedup 1.0000x reference)
import jax
import jax.numpy as jnp
from jax.experimental import pallas as pl


def kernel(memory, question, question_lengths, memory_graph, entity_table, word_table, W_enc, b_enc, W_att, W_comb, b_comb, W_score, W_proj, b_proj):
    raise NotImplementedError("write your pallas kernel here")


# speedup vs baseline: 1.3902x; 1.3902x over previous
import functools

import jax
import jax.numpy as jnp
from jax import lax
from jax.experimental import pallas as pl
from jax.experimental.pallas import tpu as pltpu
from jax.experimental.pallas import tpu_sc as plsc

_V = 100000
_E = 128
_H2 = 1024
_B = 64
_S = 50
_M = 32
_VT = 5120
_NV = (_V + _VT - 1) // _VT
_NW = 32


def _sc_mesh():
    return plsc.VectorSubcoreMesh(core_axis_name="c", subcore_axis_name="s")


def _sc_initial_gather(word_table, qidx_pad, entity_table, mem_flat):
    @functools.partial(
        pl.kernel,
        out_type=(
            jax.ShapeDtypeStruct((_NW * 104, _E), jnp.float32),
            jax.ShapeDtypeStruct((_B * _M, _E), jnp.float32),
        ),
        mesh=_sc_mesh(),
        scratch_types=[
            pltpu.VMEM((128,), jnp.int32),
            pltpu.VMEM((128, _E), jnp.float32),
            pltpu.VMEM((64,), jnp.int32),
            pltpu.VMEM((64, _E), jnp.float32),
            pltpu.SemaphoreType.DMA,
        ],
    )
    def k(wt, qi, et, mf, qe_out, me_out, qi_v, qrows_v, mi_v, mrows_v, sem):
        wid = lax.axis_index("s") * 2 + lax.axis_index("c")
        pltpu.sync_copy(qi.at[wid], qi_v)
        pltpu.async_copy(wt.at[qi_v], qrows_v, sem).wait()
        pltpu.sync_copy(qrows_v.at[pl.ds(0, 104)], qe_out.at[pl.ds(wid * 104, 104)])
        pltpu.sync_copy(mf.at[pl.ds(wid * 64, 64)], mi_v)
        pltpu.async_copy(et.at[mi_v], mrows_v, sem).wait()
        pltpu.sync_copy(mrows_v, me_out.at[pl.ds(wid * 64, 64)])

    return k(word_table, qidx_pad, entity_table, mem_flat)


def _sc_walk_gather(entity_table, mg_flat, entities):
    @functools.partial(
        pl.kernel,
        out_type=(
            jax.ShapeDtypeStruct((_B, _E), jnp.float32),
            jax.ShapeDtypeStruct((_B * _M, _E), jnp.float32),
        ),
        mesh=_sc_mesh(),
        scratch_types=[
            pltpu.VMEM((_B + 16,), jnp.int32),
            pltpu.VMEM((64,), jnp.int32),
            pltpu.VMEM((64,), jnp.int32),
            pltpu.VMEM((64, _E), jnp.float32),
            pltpu.VMEM((_B, _E), jnp.float32),
            pltpu.SemaphoreType.DMA,
        ],
    )
    def k(et, mg, ents, eemb_out, memb_out, ents_v, idx_v, mids_v, erows_v, eerows_v, sem):
        wid = lax.axis_index("s") * 2 + lax.axis_index("c")
        b0 = wid * 2
        pltpu.sync_copy(ents, ents_v.at[pl.ds(0, _B)])
        lane = lax.iota(jnp.int32, 16)
        for r in range(2):
            ent_vec = jnp.full((16,), ents_v[pl.ds(b0 + r, 16)][0], jnp.int32)
            for c in range(2):
                idx_v[pl.ds((r * 2 + c) * 16, 16)] = ent_vec * _M + (lane + c * 16)
        pltpu.async_copy(mg.at[idx_v], mids_v, sem).wait()
        pltpu.async_copy(et.at[mids_v], erows_v, sem).wait()
        pltpu.sync_copy(erows_v, memb_out.at[pl.ds(b0 * _M, 64)])

        @pl.when(wid < 8)
        def _():
            pltpu.async_copy(et.at[ents_v.at[pl.ds(0, _B)]], eerows_v, sem).wait()
            pltpu.sync_copy(eerows_v.at[pl.ds(wid * 8, 8)],
                            eemb_out.at[pl.ds(wid * 8, 8)])

    return k(entity_table, mg_flat, entities)


def _front(aq, me, watt, wca, wcb, bc, ws):
    t = lax.dot_general(aq, watt, (((1,), (1,)), ((), ())),
                        preferred_element_type=jnp.float32)
    scores = jnp.sum(me * t[:, None, :], axis=2)
    scores = scores - jnp.max(scores, axis=1, keepdims=True)
    ex = jnp.exp(scores)
    w = ex / jnp.sum(ex, axis=1, keepdims=True)
    mv = jnp.sum(w[:, :, None] * me, axis=1)
    comb = jnp.tanh(
        jnp.dot(aq, wca, preferred_element_type=jnp.float32)
        + jnp.dot(mv, wcb, preferred_element_type=jnp.float32)
        + bc)
    s = jnp.dot(comb, ws, preferred_element_type=jnp.float32)
    return comb, s


def _select_tail(i, s_scr, et_ref, g_ref, bv, bi, ent_out):
    lg = lax.dot_general(s_scr[...], et_ref[...], (((1,), (1,)), ((), ())),
                         preferred_element_type=jnp.float32) + g_ref[...]
    iota_v = lax.broadcasted_iota(jnp.int32, (_B, _VT), 1)
    lg = jnp.where(iota_v + i * _VT < _V, lg, -jnp.inf)
    mval = jnp.max(lg, axis=1, keepdims=True)
    idx = jnp.min(jnp.where(lg == mval, iota_v, _VT), axis=1, keepdims=True)
    gidx = idx + i * _VT
    upd = mval > bv[...]
    bi[...] = jnp.where(upd, gidx, bi[...])
    bv[...] = jnp.where(upd, mval, bv[...])

    @pl.when(i == _NV - 1)
    def _():
        ent_out[...] = bi[...]


def _phase0_call(q_emb3, len_f, W_enc, b_enc2, W_att, W_comb_a, W_comb_b,
                 b_comb2, W_score, mem_emb3, entity_table, g):
    def body(q_ref, len_ref, wenc, benc, watt, wca, wcb, bc, ws, me, et_ref,
             g_ref, comb_out, ent_out, s_scr, bv, bi):
        i = pl.program_id(0)

        @pl.when(i == 0)
        def _():
            iota_s = lax.broadcasted_iota(jnp.int32, (_B, _S), 1).astype(jnp.float32)
            mask = (iota_s < len_ref[...]).astype(jnp.float32)
            qp = jnp.sum(q_ref[...] * mask[:, :, None], axis=1) / len_ref[...]
            aq = jnp.tanh(jnp.dot(qp, wenc[...],
                                  preferred_element_type=jnp.float32) + benc[...])
            comb, s = _front(aq, me[...], watt[...], wca[...], wcb[...],
                             bc[...], ws[...])
            comb_out[...] = comb
            s_scr[...] = s
            bv[...] = jnp.full((_B, 1), -jnp.inf, jnp.float32)
            bi[...] = jnp.zeros((_B, 1), jnp.int32)

        _select_tail(i, s_scr, et_ref, g_ref, bv, bi, ent_out)

    full = lambda shape: pl.BlockSpec(shape, lambda i: (0,) * len(shape))
    return pl.pallas_call(
        body,
        grid=(_NV,),
        in_specs=[
            full((_B, _S, _E)),
            full((_B, 1)),
            full((_E, _H2)),
            full((1, _H2)),
            full((_E, _H2)),
            full((_H2, _H2)),
            full((_E, _H2)),
            full((1, _H2)),
            full((_H2, _E)),
            full((_B, _M, _E)),
            pl.BlockSpec((_VT, _E), lambda i: (i, 0)),
            pl.BlockSpec((_B, _VT), lambda i: (0, i)),
        ],
        out_specs=[full((_B, _H2)), full((_B, 1))],
        out_shape=[
            jax.ShapeDtypeStruct((_B, _H2), jnp.float32),
            jax.ShapeDtypeStruct((_B, 1), jnp.int32),
        ],
        scratch_shapes=[
            pltpu.VMEM((_B, _E), jnp.float32),
            pltpu.VMEM((_B, 1), jnp.float32),
            pltpu.VMEM((_B, 1), jnp.int32),
        ],
        compiler_params=pltpu.CompilerParams(
            dimension_semantics=("arbitrary",)),
    )(q_emb3, len_f, W_enc, b_enc2, W_att, W_comb_a, W_comb_b, b_comb2,
      W_score, mem_emb3, entity_table, g)


def _phase_walk_call(comb_prev, eemb, W_proj_a, W_proj_b, b_proj2, W_att,
                     W_comb_a, W_comb_b, b_comb2, W_score, mem_emb3,
                     entity_table, g):
    def body(cp, ee, wpa, wpb, bp, watt, wca, wcb, bc, ws, me, et_ref, g_ref,
             comb_out, ent_out, s_scr, bv, bi):
        i = pl.program_id(0)

        @pl.when(i == 0)
        def _():
            aq = jnp.tanh(
                jnp.dot(cp[...], wpa[...], preferred_element_type=jnp.float32)
                + jnp.dot(ee[...], wpb[...], preferred_element_type=jnp.float32)
                + bp[...])
            comb, s = _front(aq, me[...], watt[...], wca[...], wcb[...],
                             bc[...], ws[...])
            comb_out[...] = comb
            s_scr[...] = s
            bv[...] = jnp.full((_B, 1), -jnp.inf, jnp.float32)
            bi[...] = jnp.zeros((_B, 1), jnp.int32)

        _select_tail(i, s_scr, et_ref, g_ref, bv, bi, ent_out)

    full = lambda shape: pl.BlockSpec(shape, lambda i: (0,) * len(shape))
    return pl.pallas_call(
        body,
        grid=(_NV,),
        in_specs=[
            full((_B, _H2)),
            full((_B, _E)),
            full((_H2, _H2)),
            full((_E, _H2)),
            full((1, _H2)),
            full((_E, _H2)),
            full((_H2, _H2)),
            full((_E, _H2)),
            full((1, _H2)),
            full((_H2, _E)),
            full((_B, _M, _E)),
            pl.BlockSpec((_VT, _E), lambda i: (i, 0)),
            pl.BlockSpec((_B, _VT), lambda i: (0, i)),
        ],
        out_specs=[full((_B, _H2)), full((_B, 1))],
        out_shape=[
            jax.ShapeDtypeStruct((_B, _H2), jnp.float32),
            jax.ShapeDtypeStruct((_B, 1), jnp.int32),
        ],
        scratch_shapes=[
            pltpu.VMEM((_B, _E), jnp.float32),
            pltpu.VMEM((_B, 1), jnp.float32),
            pltpu.VMEM((_B, 1), jnp.int32),
        ],
        compiler_params=pltpu.CompilerParams(
            dimension_semantics=("arbitrary",)),
    )(comb_prev, eemb, W_proj_a, W_proj_b, b_proj2, W_att, W_comb_a,
      W_comb_b, b_comb2, W_score, mem_emb3, entity_table, g)


def _phase_final_call(comb_prev, eemb, W_proj_a, W_proj_b, b_proj2, W_att,
                      W_comb_a, W_comb_b, b_comb2, W_score, mem_emb3,
                      entity_table):
    def body(cp, ee, wpa, wpb, bp, watt, wca, wcb, bc, ws, me, et_ref,
             out_ref, s_scr):
        i = pl.program_id(0)

        @pl.when(i == 0)
        def _():
            aq = jnp.tanh(
                jnp.dot(cp[...], wpa[...], preferred_element_type=jnp.float32)
                + jnp.dot(ee[...], wpb[...], preferred_element_type=jnp.float32)
                + bp[...])
            _, s = _front(aq, me[...], watt[...], wca[...], wcb[...],
                          bc[...], ws[...])
            s_scr[...] = s

        out_ref[...] = lax.dot_general(
            s_scr[...], et_ref[...], (((1,), (1,)), ((), ())),
            preferred_element_type=jnp.float32)

    full = lambda shape: pl.BlockSpec(shape, lambda i: (0,) * len(shape))
    return pl.pallas_call(
        body,
        grid=(_NV,),
        in_specs=[
            full((_B, _H2)),
            full((_B, _E)),
            full((_H2, _H2)),
            full((_E, _H2)),
            full((1, _H2)),
            full((_E, _H2)),
            full((_H2, _H2)),
            full((_E, _H2)),
            full((1, _H2)),
            full((_H2, _E)),
            full((_B, _M, _E)),
            pl.BlockSpec((_VT, _E), lambda i: (i, 0)),
        ],
        out_specs=pl.BlockSpec((_B, _VT), lambda i: (0, i)),
        out_shape=jax.ShapeDtypeStruct((_B, _V), jnp.float32),
        scratch_shapes=[pltpu.VMEM((_B, _E), jnp.float32)],
        compiler_params=pltpu.CompilerParams(
            dimension_semantics=("arbitrary",)),
    )(comb_prev, eemb, W_proj_a, W_proj_b, b_proj2, W_att, W_comb_a,
      W_comb_b, b_comb2, W_score, mem_emb3, entity_table)


def kernel(memory, question, question_lengths, memory_graph, entity_table,
           word_table, W_enc, b_enc, W_att, W_comb, b_comb, W_score, W_proj,
           b_proj):
    memory = memory.astype(jnp.int32)
    question = question.astype(jnp.int32)
    memory_graph = memory_graph.astype(jnp.int32)

    len_f = jnp.maximum(question_lengths, 1).astype(jnp.float32).reshape(_B, 1)
    b_enc2 = b_enc.reshape(1, _H2)
    b_comb2 = b_comb.reshape(1, _H2)
    b_proj2 = b_proj.reshape(1, _H2)
    W_comb_a = W_comb[:_H2]
    W_comb_b = W_comb[_H2:]
    W_proj_a = W_proj[:_H2]
    W_proj_b = W_proj[_H2:]

    g0 = jax.random.gumbel(jax.random.fold_in(jax.random.key(42), 0),
                           (_B, _V), jnp.float32)
    g1 = jax.random.gumbel(jax.random.fold_in(jax.random.key(42), 1),
                           (_B, _V), jnp.float32)

    qflat = jnp.pad(question.reshape(_B * _S), (0, _NW * 104 - _B * _S))
    qidx_pad = jnp.pad(qflat.reshape(_NW, 104), ((0, 0), (0, 24)))
    mem_flat = memory.reshape(_B * _M)
    mg_flat = memory_graph.reshape(_V * _M)

    q_emb, mem_emb0 = _sc_initial_gather(word_table, qidx_pad, entity_table,
                                         mem_flat)
    comb0, ent0 = _phase0_call(
        q_emb[:_B * _S].reshape(_B, _S, _E), len_f, W_enc, b_enc2, W_att, W_comb_a,
        W_comb_b, b_comb2, W_score, mem_emb0.reshape(_B, _M, _E),
        entity_table, g0)

    eemb1, mem_emb1 = _sc_walk_gather(entity_table, mg_flat, ent0.reshape(_B))
    comb1, ent1 = _phase_walk_call(
        comb0, eemb1, W_proj_a, W_proj_b, b_proj2, W_att, W_comb_a, W_comb_b,
        b_comb2, W_score, mem_emb1.reshape(_B, _M, _E), entity_table, g1)

    eemb2, mem_emb2 = _sc_walk_gather(entity_table, mg_flat, ent1.reshape(_B))
    logits = _phase_final_call(
        comb1, eemb2, W_proj_a, W_proj_b, b_proj2, W_att, W_comb_a, W_comb_b,
        b_comb2, W_score, mem_emb2.reshape(_B, _M, _E), entity_table)
    return logits


# speedup vs baseline: 1.3981x; 1.0056x over previous
import functools

import jax
import jax.numpy as jnp
from jax import lax
from jax.experimental import pallas as pl
from jax.experimental.pallas import tpu as pltpu
from jax.experimental.pallas import tpu_sc as plsc

_V = 100000
_E = 128
_H2 = 1024
_B = 64
_S = 50
_M = 32
_VT = 5120
_NV = (_V + _VT - 1) // _VT
_NW = 32


def _sc_mesh():
    return plsc.VectorSubcoreMesh(core_axis_name="c", subcore_axis_name="s")


def _sc_initial_gather(word_table, qidx_pad, entity_table, mem_flat):
    @functools.partial(
        pl.kernel,
        out_type=(
            jax.ShapeDtypeStruct((_NW * 104, _E), jnp.float32),
            jax.ShapeDtypeStruct((_B * _M, _E), jnp.float32),
        ),
        mesh=_sc_mesh(),
        scratch_types=[
            pltpu.VMEM((128,), jnp.int32),
            pltpu.VMEM((128, _E), jnp.float32),
            pltpu.VMEM((64,), jnp.int32),
            pltpu.VMEM((64, _E), jnp.float32),
            pltpu.SemaphoreType.DMA,
        ],
    )
    def k(wt, qi, et, mf, qe_out, me_out, qi_v, qrows_v, mi_v, mrows_v, sem):
        wid = lax.axis_index("s") * 2 + lax.axis_index("c")
        ca = pltpu.async_copy(qi.at[wid], qi_v, sem)
        cb = pltpu.async_copy(mf.at[pl.ds(wid * 64, 64)], mi_v, sem)
        ca.wait()
        cb.wait()
        cc = pltpu.async_copy(wt.at[qi_v], qrows_v, sem)
        cd = pltpu.async_copy(et.at[mi_v], mrows_v, sem)
        cc.wait()
        cd.wait()
        ce = pltpu.async_copy(qrows_v.at[pl.ds(0, 104)],
                              qe_out.at[pl.ds(wid * 104, 104)], sem)
        cf = pltpu.async_copy(mrows_v, me_out.at[pl.ds(wid * 64, 64)], sem)
        ce.wait()
        cf.wait()

    return k(word_table, qidx_pad, entity_table, mem_flat)


def _sc_walk_gather(entity_table, mg_flat, entities):
    @functools.partial(
        pl.kernel,
        out_type=(
            jax.ShapeDtypeStruct((_B, _E), jnp.float32),
            jax.ShapeDtypeStruct((_B * _M, _E), jnp.float32),
        ),
        mesh=_sc_mesh(),
        scratch_types=[
            pltpu.VMEM((_B + 16,), jnp.int32),
            pltpu.VMEM((64,), jnp.int32),
            pltpu.VMEM((64,), jnp.int32),
            pltpu.VMEM((64, _E), jnp.float32),
            pltpu.VMEM((_B, _E), jnp.float32),
            pltpu.SemaphoreType.DMA,
            pltpu.SemaphoreType.DMA,
        ],
    )
    def k(et, mg, ents, eemb_out, memb_out, ents_v, idx_v, mids_v, erows_v,
          eerows_v, sem, sem2):
        wid = lax.axis_index("s") * 2 + lax.axis_index("c")
        b0 = wid * 2
        pltpu.sync_copy(ents, ents_v.at[pl.ds(0, _B)])
        @pl.when(wid < 8)
        def _():
            pltpu.async_copy(et.at[ents_v.at[pl.ds(0, _B)]], eerows_v, sem2)

        lane = lax.iota(jnp.int32, 16)
        for r in range(2):
            ent_vec = jnp.full((16,), ents_v[pl.ds(b0 + r, 16)][0], jnp.int32)
            for c in range(2):
                idx_v[pl.ds((r * 2 + c) * 16, 16)] = ent_vec * _M + (lane + c * 16)
        pltpu.async_copy(mg.at[idx_v], mids_v, sem).wait()
        pltpu.async_copy(et.at[mids_v], erows_v, sem).wait()
        pltpu.sync_copy(erows_v, memb_out.at[pl.ds(b0 * _M, 64)])

        @pl.when(wid < 8)
        def _():
            pltpu.make_async_copy(et.at[ents_v.at[pl.ds(0, _B)]], eerows_v,
                                  sem2).wait()
            pltpu.sync_copy(eerows_v.at[pl.ds(wid * 8, 8)],
                            eemb_out.at[pl.ds(wid * 8, 8)])

    return k(entity_table, mg_flat, entities)


def _front(aq, me, watt, wca, wcb, bc, ws):
    t = lax.dot_general(aq, watt, (((1,), (1,)), ((), ())),
                        preferred_element_type=jnp.float32)
    scores = jnp.sum(me * t[:, None, :], axis=2)
    scores = scores - jnp.max(scores, axis=1, keepdims=True)
    ex = jnp.exp(scores)
    w = ex / jnp.sum(ex, axis=1, keepdims=True)
    mv = jnp.sum(w[:, :, None] * me, axis=1)
    comb = jnp.tanh(
        jnp.dot(aq, wca, preferred_element_type=jnp.float32)
        + jnp.dot(mv, wcb, preferred_element_type=jnp.float32)
        + bc)
    s = jnp.dot(comb, ws, preferred_element_type=jnp.float32)
    return comb, s


def _select_tail(i, s_scr, et_ref, g_ref, bv, bi, ent_out):
    lg = lax.dot_general(s_scr[...], et_ref[...], (((1,), (1,)), ((), ())),
                         preferred_element_type=jnp.float32) + g_ref[...]
    iota_v = lax.broadcasted_iota(jnp.int32, (_B, _VT), 1)
    lg = jnp.where(iota_v + i * _VT < _V, lg, -jnp.inf)
    mval = jnp.max(lg, axis=1, keepdims=True)
    idx = jnp.min(jnp.where(lg == mval, iota_v, _VT), axis=1, keepdims=True)
    gidx = idx + i * _VT
    upd = mval > bv[...]
    bi[...] = jnp.where(upd, gidx, bi[...])
    bv[...] = jnp.where(upd, mval, bv[...])

    @pl.when(i == _NV - 1)
    def _():
        ent_out[...] = bi[...]


def _phase0_call(q_emb3, len_f, W_enc, b_enc2, W_att, W_comb_a, W_comb_b,
                 b_comb2, W_score, mem_emb3, entity_table, g):
    def body(q_ref, len_ref, wenc, benc, watt, wca, wcb, bc, ws, me, et_ref,
             g_ref, comb_out, ent_out, s_scr, bv, bi):
        i = pl.program_id(0)

        @pl.when(i == 0)
        def _():
            iota_s = lax.broadcasted_iota(jnp.int32, (_B, _S), 1).astype(jnp.float32)
            mask = (iota_s < len_ref[...]).astype(jnp.float32)
            qp = jnp.sum(q_ref[...] * mask[:, :, None], axis=1) / len_ref[...]
            aq = jnp.tanh(jnp.dot(qp, wenc[...],
                                  preferred_element_type=jnp.float32) + benc[...])
            comb, s = _front(aq, me[...], watt[...], wca[...], wcb[...],
                             bc[...], ws[...])
            comb_out[...] = comb
            s_scr[...] = s
            bv[...] = jnp.full((_B, 1), -jnp.inf, jnp.float32)
            bi[...] = jnp.zeros((_B, 1), jnp.int32)

        _select_tail(i, s_scr, et_ref, g_ref, bv, bi, ent_out)

    full = lambda shape: pl.BlockSpec(shape, lambda i: (0,) * len(shape))
    return pl.pallas_call(
        body,
        grid=(_NV,),
        in_specs=[
            full((_B, _S, _E)),
            full((_B, 1)),
            full((_E, _H2)),
            full((1, _H2)),
            full((_E, _H2)),
            full((_H2, _H2)),
            full((_E, _H2)),
            full((1, _H2)),
            full((_H2, _E)),
            full((_B, _M, _E)),
            pl.BlockSpec((_VT, _E), lambda i: (i, 0)),
            pl.BlockSpec((_B, _VT), lambda i: (0, i)),
        ],
        out_specs=[full((_B, _H2)), full((_B, 1))],
        out_shape=[
            jax.ShapeDtypeStruct((_B, _H2), jnp.float32),
            jax.ShapeDtypeStruct((_B, 1), jnp.int32),
        ],
        scratch_shapes=[
            pltpu.VMEM((_B, _E), jnp.float32),
            pltpu.VMEM((_B, 1), jnp.float32),
            pltpu.VMEM((_B, 1), jnp.int32),
        ],
        compiler_params=pltpu.CompilerParams(
            dimension_semantics=("arbitrary",)),
    )(q_emb3, len_f, W_enc, b_enc2, W_att, W_comb_a, W_comb_b, b_comb2,
      W_score, mem_emb3, entity_table, g)


def _phase_walk_call(comb_prev, eemb, W_proj_a, W_proj_b, b_proj2, W_att,
                     W_comb_a, W_comb_b, b_comb2, W_score, mem_emb3,
                     entity_table, g):
    def body(cp, ee, wpa, wpb, bp, watt, wca, wcb, bc, ws, me, et_ref, g_ref,
             comb_out, ent_out, s_scr, bv, bi):
        i = pl.program_id(0)

        @pl.when(i == 0)
        def _():
            aq = jnp.tanh(
                jnp.dot(cp[...], wpa[...], preferred_element_type=jnp.float32)
                + jnp.dot(ee[...], wpb[...], preferred_element_type=jnp.float32)
                + bp[...])
            comb, s = _front(aq, me[...], watt[...], wca[...], wcb[...],
                             bc[...], ws[...])
            comb_out[...] = comb
            s_scr[...] = s
            bv[...] = jnp.full((_B, 1), -jnp.inf, jnp.float32)
            bi[...] = jnp.zeros((_B, 1), jnp.int32)

        _select_tail(i, s_scr, et_ref, g_ref, bv, bi, ent_out)

    full = lambda shape: pl.BlockSpec(shape, lambda i: (0,) * len(shape))
    return pl.pallas_call(
        body,
        grid=(_NV,),
        in_specs=[
            full((_B, _H2)),
            full((_B, _E)),
            full((_H2, _H2)),
            full((_E, _H2)),
            full((1, _H2)),
            full((_E, _H2)),
            full((_H2, _H2)),
            full((_E, _H2)),
            full((1, _H2)),
            full((_H2, _E)),
            full((_B, _M, _E)),
            pl.BlockSpec((_VT, _E), lambda i: (i, 0)),
            pl.BlockSpec((_B, _VT), lambda i: (0, i)),
        ],
        out_specs=[full((_B, _H2)), full((_B, 1))],
        out_shape=[
            jax.ShapeDtypeStruct((_B, _H2), jnp.float32),
            jax.ShapeDtypeStruct((_B, 1), jnp.int32),
        ],
        scratch_shapes=[
            pltpu.VMEM((_B, _E), jnp.float32),
            pltpu.VMEM((_B, 1), jnp.float32),
            pltpu.VMEM((_B, 1), jnp.int32),
        ],
        compiler_params=pltpu.CompilerParams(
            dimension_semantics=("arbitrary",)),
    )(comb_prev, eemb, W_proj_a, W_proj_b, b_proj2, W_att, W_comb_a,
      W_comb_b, b_comb2, W_score, mem_emb3, entity_table, g)


def _phase_final_call(comb_prev, eemb, W_proj_a, W_proj_b, b_proj2, W_att,
                      W_comb_a, W_comb_b, b_comb2, W_score, mem_emb3,
                      entity_table):
    def body(cp, ee, wpa, wpb, bp, watt, wca, wcb, bc, ws, me, et_ref,
             out_ref, s_scr):
        i = pl.program_id(0)

        @pl.when(i == 0)
        def _():
            aq = jnp.tanh(
                jnp.dot(cp[...], wpa[...], preferred_element_type=jnp.float32)
                + jnp.dot(ee[...], wpb[...], preferred_element_type=jnp.float32)
                + bp[...])
            _, s = _front(aq, me[...], watt[...], wca[...], wcb[...],
                          bc[...], ws[...])
            s_scr[...] = s

        out_ref[...] = lax.dot_general(
            s_scr[...], et_ref[...], (((1,), (1,)), ((), ())),
            preferred_element_type=jnp.float32)

    full = lambda shape: pl.BlockSpec(shape, lambda i: (0,) * len(shape))
    return pl.pallas_call(
        body,
        grid=(_NV,),
        in_specs=[
            full((_B, _H2)),
            full((_B, _E)),
            full((_H2, _H2)),
            full((_E, _H2)),
            full((1, _H2)),
            full((_E, _H2)),
            full((_H2, _H2)),
            full((_E, _H2)),
            full((1, _H2)),
            full((_H2, _E)),
            full((_B, _M, _E)),
            pl.BlockSpec((_VT, _E), lambda i: (i, 0)),
        ],
        out_specs=pl.BlockSpec((_B, _VT), lambda i: (0, i)),
        out_shape=jax.ShapeDtypeStruct((_B, _V), jnp.float32),
        scratch_shapes=[pltpu.VMEM((_B, _E), jnp.float32)],
        compiler_params=pltpu.CompilerParams(
            dimension_semantics=("arbitrary",)),
    )(comb_prev, eemb, W_proj_a, W_proj_b, b_proj2, W_att, W_comb_a,
      W_comb_b, b_comb2, W_score, mem_emb3, entity_table)


def kernel(memory, question, question_lengths, memory_graph, entity_table,
           word_table, W_enc, b_enc, W_att, W_comb, b_comb, W_score, W_proj,
           b_proj):
    memory = memory.astype(jnp.int32)
    question = question.astype(jnp.int32)
    memory_graph = memory_graph.astype(jnp.int32)

    len_f = jnp.maximum(question_lengths, 1).astype(jnp.float32).reshape(_B, 1)
    b_enc2 = b_enc.reshape(1, _H2)
    b_comb2 = b_comb.reshape(1, _H2)
    b_proj2 = b_proj.reshape(1, _H2)
    W_comb_a = W_comb[:_H2]
    W_comb_b = W_comb[_H2:]
    W_proj_a = W_proj[:_H2]
    W_proj_b = W_proj[_H2:]

    g0 = jax.random.gumbel(jax.random.fold_in(jax.random.key(42), 0),
                           (_B, _V), jnp.float32)
    g1 = jax.random.gumbel(jax.random.fold_in(jax.random.key(42), 1),
                           (_B, _V), jnp.float32)

    qflat = jnp.pad(question.reshape(_B * _S), (0, _NW * 104 - _B * _S))
    qidx_pad = jnp.pad(qflat.reshape(_NW, 104), ((0, 0), (0, 24)))
    mem_flat = memory.reshape(_B * _M)
    mg_flat = memory_graph.reshape(_V * _M)

    q_emb, mem_emb0 = _sc_initial_gather(word_table, qidx_pad, entity_table,
                                         mem_flat)
    comb0, ent0 = _phase0_call(
        q_emb[:_B * _S].reshape(_B, _S, _E), len_f, W_enc, b_enc2, W_att, W_comb_a,
        W_comb_b, b_comb2, W_score, mem_emb0.reshape(_B, _M, _E),
        entity_table, g0)

    eemb1, mem_emb1 = _sc_walk_gather(entity_table, mg_flat, ent0.reshape(_B))
    comb1, ent1 = _phase_walk_call(
        comb0, eemb1, W_proj_a, W_proj_b, b_proj2, W_att, W_comb_a, W_comb_b,
        b_comb2, W_score, mem_emb1.reshape(_B, _M, _E), entity_table, g1)

    eemb2, mem_emb2 = _sc_walk_gather(entity_table, mg_flat, ent1.reshape(_B))
    logits = _phase_final_call(
        comb1, eemb2, W_proj_a, W_proj_b, b_proj2, W_att, W_comb_a, W_comb_b,
        b_comb2, W_score, mem_emb2.reshape(_B, _M, _E), entity_table)
    return logits


# speedup vs baseline: 1.4385x; 1.0289x over previous
import functools

import jax
import jax.numpy as jnp
from jax import lax
from jax.experimental import pallas as pl
from jax.experimental.pallas import tpu as pltpu
from jax.experimental.pallas import tpu_sc as plsc

_V = 100000
_E = 128
_H2 = 1024
_B = 64
_S = 50
_M = 32
_VT = 5120
_NV = (_V + _VT - 1) // _VT
_NW = 32


_TF_R1 = (13, 15, 26, 6)
_TF_R2 = (17, 29, 16, 24)
_TF_PARITY = 0x1BD11BDA


def _tf_py(k1, k2, x0, x1):
    msk = 0xFFFFFFFF
    rot = lambda x, r: ((x << r) | (x >> (32 - r))) & msk
    ks = [k1, k2, (k1 ^ k2 ^ _TF_PARITY) & msk]
    x0 = (x0 + ks[0]) & msk
    x1 = (x1 + ks[1]) & msk
    for j, rs in enumerate((_TF_R1, _TF_R2, _TF_R1, _TF_R2, _TF_R1)):
        for r in rs:
            x0 = (x0 + x1) & msk
            x1 = x0 ^ rot(x1, r)
        x0 = (x0 + ks[(j + 1) % 3]) & msk
        x1 = (x1 + ks[(j + 2) % 3] + j + 1) & msk
    return x0, x1


def _fold_in_py(i):
    return _tf_py(0, 42, 0, i)


def _i32c(x):
    x &= 0xFFFFFFFF
    return x - (1 << 32) if x >= (1 << 31) else x


def _tf_bits(k1, k2, p):
    c = lambda v: jnp.int32(_i32c(v))
    ks0, ks1 = k1, k2
    ks2 = k1 ^ k2 ^ _TF_PARITY

    def rot(x, r):
        return lax.shift_left(x, jnp.int32(r)) | lax.shift_right_logical(
            x, jnp.int32(32 - r))

    x0 = jnp.full(p.shape, c(ks0), jnp.int32)
    x1 = p + c(ks1)
    ks = (ks0, ks1, ks2)
    for j, rs in enumerate((_TF_R1, _TF_R2, _TF_R1, _TF_R2, _TF_R1)):
        for r in rs:
            x0 = x0 + x1
            x1 = x0 ^ rot(x1, r)
        x0 = x0 + c(ks[(j + 1) % 3])
        x1 = x1 + c(ks[(j + 2) % 3] + j + 1)
    return x0 ^ x1


_K0 = _fold_in_py(0)
_K1 = _fold_in_py(1)
_F32_TINY = 1.1754943508222875e-38


def _gumbel_tile(key_words, i):
    row = lax.broadcasted_iota(jnp.int32, (_B, _VT), 0)
    col = lax.broadcasted_iota(jnp.int32, (_B, _VT), 1) + i * _VT
    p = row * _V + col
    bits = _tf_bits(key_words[0], key_words[1], p)
    fb = lax.shift_right_logical(bits, jnp.int32(9)) | jnp.int32(0x3F800000)
    f = lax.bitcast_convert_type(fb, jnp.float32) - jnp.float32(1.0)
    u = jnp.maximum(f, jnp.float32(_F32_TINY))
    return -jnp.log(-jnp.log(u))


def _sc_mesh():
    return plsc.VectorSubcoreMesh(core_axis_name="c", subcore_axis_name="s")


def _sc_initial_gather(word_table, qidx_pad, entity_table, mem_flat):
    @functools.partial(
        pl.kernel,
        out_type=(
            jax.ShapeDtypeStruct((_NW * 104, _E), jnp.float32),
            jax.ShapeDtypeStruct((_B * _M, _E), jnp.float32),
        ),
        mesh=_sc_mesh(),
        scratch_types=[
            pltpu.VMEM((128,), jnp.int32),
            pltpu.VMEM((128, _E), jnp.float32),
            pltpu.VMEM((64,), jnp.int32),
            pltpu.VMEM((64, _E), jnp.float32),
            pltpu.SemaphoreType.DMA,
        ],
    )
    def k(wt, qi, et, mf, qe_out, me_out, qi_v, qrows_v, mi_v, mrows_v, sem):
        wid = lax.axis_index("s") * 2 + lax.axis_index("c")
        ca = pltpu.async_copy(qi.at[wid], qi_v, sem)
        cb = pltpu.async_copy(mf.at[pl.ds(wid * 64, 64)], mi_v, sem)
        ca.wait()
        cb.wait()
        cc = pltpu.async_copy(wt.at[qi_v], qrows_v, sem)
        cd = pltpu.async_copy(et.at[mi_v], mrows_v, sem)
        cc.wait()
        cd.wait()
        ce = pltpu.async_copy(qrows_v.at[pl.ds(0, 104)],
                              qe_out.at[pl.ds(wid * 104, 104)], sem)
        cf = pltpu.async_copy(mrows_v, me_out.at[pl.ds(wid * 64, 64)], sem)
        ce.wait()
        cf.wait()

    return k(word_table, qidx_pad, entity_table, mem_flat)


def _sc_walk_gather(entity_table, mg_flat, entities):
    @functools.partial(
        pl.kernel,
        out_type=(
            jax.ShapeDtypeStruct((_B, _E), jnp.float32),
            jax.ShapeDtypeStruct((_B * _M, _E), jnp.float32),
        ),
        mesh=_sc_mesh(),
        scratch_types=[
            pltpu.VMEM((_B + 16,), jnp.int32),
            pltpu.VMEM((64,), jnp.int32),
            pltpu.VMEM((64,), jnp.int32),
            pltpu.VMEM((64, _E), jnp.float32),
            pltpu.VMEM((_B, _E), jnp.float32),
            pltpu.SemaphoreType.DMA,
            pltpu.SemaphoreType.DMA,
        ],
    )
    def k(et, mg, ents, eemb_out, memb_out, ents_v, idx_v, mids_v, erows_v,
          eerows_v, sem, sem2):
        wid = lax.axis_index("s") * 2 + lax.axis_index("c")
        b0 = wid * 2
        pltpu.sync_copy(ents, ents_v.at[pl.ds(0, _B)])
        @pl.when(wid < 8)
        def _():
            pltpu.async_copy(et.at[ents_v.at[pl.ds(0, _B)]], eerows_v, sem2)

        lane = lax.iota(jnp.int32, 16)
        for r in range(2):
            ent_vec = jnp.full((16,), ents_v[pl.ds(b0 + r, 16)][0], jnp.int32)
            for c in range(2):
                idx_v[pl.ds((r * 2 + c) * 16, 16)] = ent_vec * _M + (lane + c * 16)
        pltpu.async_copy(mg.at[idx_v], mids_v, sem).wait()
        pltpu.async_copy(et.at[mids_v], erows_v, sem).wait()
        pltpu.sync_copy(erows_v, memb_out.at[pl.ds(b0 * _M, 64)])

        @pl.when(wid < 8)
        def _():
            pltpu.make_async_copy(et.at[ents_v.at[pl.ds(0, _B)]], eerows_v,
                                  sem2).wait()
            pltpu.sync_copy(eerows_v.at[pl.ds(wid * 8, 8)],
                            eemb_out.at[pl.ds(wid * 8, 8)])

    return k(entity_table, mg_flat, entities)


def _front(aq, me, watt, wca, wcb, bc, ws):
    t = lax.dot_general(aq, watt, (((1,), (1,)), ((), ())),
                        preferred_element_type=jnp.float32)
    scores = jnp.sum(me * t[:, None, :], axis=2)
    scores = scores - jnp.max(scores, axis=1, keepdims=True)
    ex = jnp.exp(scores)
    w = ex / jnp.sum(ex, axis=1, keepdims=True)
    mv = jnp.sum(w[:, :, None] * me, axis=1)
    comb = jnp.tanh(
        jnp.dot(aq, wca, preferred_element_type=jnp.float32)
        + jnp.dot(mv, wcb, preferred_element_type=jnp.float32)
        + bc)
    s = jnp.dot(comb, ws, preferred_element_type=jnp.float32)
    return comb, s


def _select_tail(i, s_scr, et_ref, key_words, bv, bi, ent_out):
    lg = lax.dot_general(s_scr[...], et_ref[...], (((1,), (1,)), ((), ())),
                         preferred_element_type=jnp.float32) + _gumbel_tile(key_words, i)
    iota_v = lax.broadcasted_iota(jnp.int32, (_B, _VT), 1)
    lg = jnp.where(iota_v + i * _VT < _V, lg, -jnp.inf)
    mval = jnp.max(lg, axis=1, keepdims=True)
    idx = jnp.min(jnp.where(lg == mval, iota_v, _VT), axis=1, keepdims=True)
    gidx = idx + i * _VT
    upd = mval > bv[...]
    bi[...] = jnp.where(upd, gidx, bi[...])
    bv[...] = jnp.where(upd, mval, bv[...])

    @pl.when(i == _NV - 1)
    def _():
        ent_out[...] = bi[...]


def _phase0_call(q_emb3, len_f, W_enc, b_enc2, W_att, W_comb, b_comb2,
                 W_score, mem_emb3, entity_table, key_words):
    def body(q_ref, len_ref, wenc, benc, watt, wc, bc, ws, me, et_ref,
             comb_out, ent_out, s_scr, bv, bi):
        i = pl.program_id(0)

        @pl.when(i == 0)
        def _():
            iota_s = lax.broadcasted_iota(jnp.int32, (_B, _S), 1).astype(jnp.float32)
            mask = (iota_s < len_ref[...]).astype(jnp.float32)
            qp = jnp.sum(q_ref[...] * mask[:, :, None], axis=1) / len_ref[...]
            aq = jnp.tanh(jnp.dot(qp, wenc[...],
                                  preferred_element_type=jnp.float32) + benc[...])
            comb, s = _front(aq, me[...], watt[...], wc[0:_H2, :],
                             wc[_H2:, :], bc[...], ws[...])
            comb_out[...] = comb
            s_scr[...] = s
            bv[...] = jnp.full((_B, 1), -jnp.inf, jnp.float32)
            bi[...] = jnp.zeros((_B, 1), jnp.int32)

        _select_tail(i, s_scr, et_ref, key_words, bv, bi, ent_out)

    full = lambda shape: pl.BlockSpec(shape, lambda i: (0,) * len(shape))
    return pl.pallas_call(
        body,
        grid=(_NV,),
        in_specs=[
            full((_B, _S, _E)),
            full((_B, 1)),
            full((_E, _H2)),
            full((1, _H2)),
            full((_E, _H2)),
            full((_H2 + _E, _H2)),
            full((1, _H2)),
            full((_H2, _E)),
            full((_B, _M, _E)),
            pl.BlockSpec((_VT, _E), lambda i: (i, 0)),
        ],
        out_specs=[full((_B, _H2)), full((_B, 1))],
        out_shape=[
            jax.ShapeDtypeStruct((_B, _H2), jnp.float32),
            jax.ShapeDtypeStruct((_B, 1), jnp.int32),
        ],
        scratch_shapes=[
            pltpu.VMEM((_B, _E), jnp.float32),
            pltpu.VMEM((_B, 1), jnp.float32),
            pltpu.VMEM((_B, 1), jnp.int32),
        ],
        compiler_params=pltpu.CompilerParams(
            dimension_semantics=("arbitrary",)),
    )(q_emb3, len_f, W_enc, b_enc2, W_att, W_comb, b_comb2,
      W_score, mem_emb3, entity_table)


def _phase_walk_call(comb_prev, eemb, W_proj, b_proj2, W_att,
                     W_comb, b_comb2, W_score, mem_emb3,
                     entity_table, key_words):
    def body(cp, ee, wp, bp, watt, wc, bc, ws, me, et_ref,
             comb_out, ent_out, s_scr, bv, bi):
        i = pl.program_id(0)

        @pl.when(i == 0)
        def _():
            aq = jnp.tanh(
                jnp.dot(cp[...], wp[0:_H2, :], preferred_element_type=jnp.float32)
                + jnp.dot(ee[...], wp[_H2:, :], preferred_element_type=jnp.float32)
                + bp[...])
            comb, s = _front(aq, me[...], watt[...], wc[0:_H2, :],
                             wc[_H2:, :], bc[...], ws[...])
            comb_out[...] = comb
            s_scr[...] = s
            bv[...] = jnp.full((_B, 1), -jnp.inf, jnp.float32)
            bi[...] = jnp.zeros((_B, 1), jnp.int32)

        _select_tail(i, s_scr, et_ref, key_words, bv, bi, ent_out)

    full = lambda shape: pl.BlockSpec(shape, lambda i: (0,) * len(shape))
    return pl.pallas_call(
        body,
        grid=(_NV,),
        in_specs=[
            full((_B, _H2)),
            full((_B, _E)),
            full((_H2 + _E, _H2)),
            full((1, _H2)),
            full((_E, _H2)),
            full((_H2 + _E, _H2)),
            full((1, _H2)),
            full((_H2, _E)),
            full((_B, _M, _E)),
            pl.BlockSpec((_VT, _E), lambda i: (i, 0)),
        ],
        out_specs=[full((_B, _H2)), full((_B, 1))],
        out_shape=[
            jax.ShapeDtypeStruct((_B, _H2), jnp.float32),
            jax.ShapeDtypeStruct((_B, 1), jnp.int32),
        ],
        scratch_shapes=[
            pltpu.VMEM((_B, _E), jnp.float32),
            pltpu.VMEM((_B, 1), jnp.float32),
            pltpu.VMEM((_B, 1), jnp.int32),
        ],
        compiler_params=pltpu.CompilerParams(
            dimension_semantics=("arbitrary",)),
    )(comb_prev, eemb, W_proj, b_proj2, W_att, W_comb,
      b_comb2, W_score, mem_emb3, entity_table)


def _phase_final_call(comb_prev, eemb, W_proj, b_proj2, W_att,
                      W_comb, b_comb2, W_score, mem_emb3,
                      entity_table):
    def body(cp, ee, wp, bp, watt, wc, bc, ws, me, et_ref,
             out_ref, s_scr):
        i = pl.program_id(0)

        @pl.when(i == 0)
        def _():
            aq = jnp.tanh(
                jnp.dot(cp[...], wp[0:_H2, :], preferred_element_type=jnp.float32)
                + jnp.dot(ee[...], wp[_H2:, :], preferred_element_type=jnp.float32)
                + bp[...])
            _, s = _front(aq, me[...], watt[...], wc[0:_H2, :],
                          wc[_H2:, :], bc[...], ws[...])
            s_scr[...] = s

        out_ref[...] = lax.dot_general(
            s_scr[...], et_ref[...], (((1,), (1,)), ((), ())),
            preferred_element_type=jnp.float32)

    full = lambda shape: pl.BlockSpec(shape, lambda i: (0,) * len(shape))
    return pl.pallas_call(
        body,
        grid=(_NV,),
        in_specs=[
            full((_B, _H2)),
            full((_B, _E)),
            full((_H2 + _E, _H2)),
            full((1, _H2)),
            full((_E, _H2)),
            full((_H2 + _E, _H2)),
            full((1, _H2)),
            full((_H2, _E)),
            full((_B, _M, _E)),
            pl.BlockSpec((_VT, _E), lambda i: (i, 0)),
        ],
        out_specs=pl.BlockSpec((_B, _VT), lambda i: (0, i)),
        out_shape=jax.ShapeDtypeStruct((_B, _V), jnp.float32),
        scratch_shapes=[pltpu.VMEM((_B, _E), jnp.float32)],
        compiler_params=pltpu.CompilerParams(
            dimension_semantics=("arbitrary",)),
    )(comb_prev, eemb, W_proj, b_proj2, W_att, W_comb,
      b_comb2, W_score, mem_emb3, entity_table)


def kernel(memory, question, question_lengths, memory_graph, entity_table,
           word_table, W_enc, b_enc, W_att, W_comb, b_comb, W_score, W_proj,
           b_proj):
    memory = memory.astype(jnp.int32)
    question = question.astype(jnp.int32)
    memory_graph = memory_graph.astype(jnp.int32)

    len_f = jnp.maximum(question_lengths, 1).astype(jnp.float32).reshape(_B, 1)
    b_enc2 = b_enc.reshape(1, _H2)
    b_comb2 = b_comb.reshape(1, _H2)
    b_proj2 = b_proj.reshape(1, _H2)

    qflat = jnp.pad(question.reshape(_B * _S), (0, _NW * 104 - _B * _S))
    qidx_pad = jnp.pad(qflat.reshape(_NW, 104), ((0, 0), (0, 24)))
    mem_flat = memory.reshape(_B * _M)
    mg_flat = memory_graph.reshape(_V * _M)

    q_emb, mem_emb0 = _sc_initial_gather(word_table, qidx_pad, entity_table,
                                         mem_flat)
    comb0, ent0 = _phase0_call(
        q_emb[:_B * _S].reshape(_B, _S, _E), len_f, W_enc, b_enc2, W_att,
        W_comb, b_comb2, W_score, mem_emb0.reshape(_B, _M, _E),
        entity_table, _K0)

    eemb1, mem_emb1 = _sc_walk_gather(entity_table, mg_flat, ent0.reshape(_B))
    comb1, ent1 = _phase_walk_call(
        comb0, eemb1, W_proj, b_proj2, W_att, W_comb,
        b_comb2, W_score, mem_emb1.reshape(_B, _M, _E), entity_table, _K1)

    eemb2, mem_emb2 = _sc_walk_gather(entity_table, mg_flat, ent1.reshape(_B))
    logits = _phase_final_call(
        comb1, eemb2, W_proj, b_proj2, W_att, W_comb,
        b_comb2, W_score, mem_emb2.reshape(_B, _M, _E), entity_table)
    return logits


# speedup vs baseline: 1.4570x; 1.0128x over previous
import functools

import jax
import jax.numpy as jnp
from jax import lax
from jax.experimental import pallas as pl
from jax.experimental.pallas import tpu as pltpu
from jax.experimental.pallas import tpu_sc as plsc

_V = 100000
_E = 128
_H2 = 1024
_B = 64
_S = 50
_M = 32
_VT = 5120
_NV = (_V + _VT - 1) // _VT
_NW = 32


_TF_R1 = (13, 15, 26, 6)
_TF_R2 = (17, 29, 16, 24)
_TF_PARITY = 0x1BD11BDA


def _tf_py(k1, k2, x0, x1):
    msk = 0xFFFFFFFF
    rot = lambda x, r: ((x << r) | (x >> (32 - r))) & msk
    ks = [k1, k2, (k1 ^ k2 ^ _TF_PARITY) & msk]
    x0 = (x0 + ks[0]) & msk
    x1 = (x1 + ks[1]) & msk
    for j, rs in enumerate((_TF_R1, _TF_R2, _TF_R1, _TF_R2, _TF_R1)):
        for r in rs:
            x0 = (x0 + x1) & msk
            x1 = x0 ^ rot(x1, r)
        x0 = (x0 + ks[(j + 1) % 3]) & msk
        x1 = (x1 + ks[(j + 2) % 3] + j + 1) & msk
    return x0, x1


def _fold_in_py(i):
    return _tf_py(0, 42, 0, i)


def _i32c(x):
    x &= 0xFFFFFFFF
    return x - (1 << 32) if x >= (1 << 31) else x


def _tf_bits(k1, k2, p):
    c = lambda v: jnp.int32(_i32c(v))
    ks0, ks1 = k1, k2
    ks2 = k1 ^ k2 ^ _TF_PARITY

    def rot(x, r):
        return lax.shift_left(x, jnp.int32(r)) | lax.shift_right_logical(
            x, jnp.int32(32 - r))

    x0 = jnp.full(p.shape, c(ks0), jnp.int32)
    x1 = p + c(ks1)
    ks = (ks0, ks1, ks2)
    for j, rs in enumerate((_TF_R1, _TF_R2, _TF_R1, _TF_R2, _TF_R1)):
        for r in rs:
            x0 = x0 + x1
            x1 = x0 ^ rot(x1, r)
        x0 = x0 + c(ks[(j + 1) % 3])
        x1 = x1 + c(ks[(j + 2) % 3] + j + 1)
    return x0 ^ x1


_K0 = _fold_in_py(0)
_K1 = _fold_in_py(1)
_F32_TINY = 1.1754943508222875e-38


def _gumbel_tile(key_words, i):
    row = lax.broadcasted_iota(jnp.int32, (_B, _VT), 0)
    col = lax.broadcasted_iota(jnp.int32, (_B, _VT), 1) + i * _VT
    p = row * _V + col
    bits = _tf_bits(key_words[0], key_words[1], p)
    fb = lax.shift_right_logical(bits, jnp.int32(9)) | jnp.int32(0x3F800000)
    f = lax.bitcast_convert_type(fb, jnp.float32) - jnp.float32(1.0)
    u = jnp.maximum(f, jnp.float32(_F32_TINY))
    return -jnp.log(-jnp.log(u))


def _sc_mesh():
    return plsc.VectorSubcoreMesh(core_axis_name="c", subcore_axis_name="s")


def _sc_initial_gather(word_table, qidx_pad, entity_table, mem_flat):
    @functools.partial(
        pl.kernel,
        out_type=(
            jax.ShapeDtypeStruct((_NW * 104, _E), jnp.float32),
            jax.ShapeDtypeStruct((_B * _M, _E), jnp.float32),
        ),
        mesh=_sc_mesh(),
        scratch_types=[
            pltpu.VMEM((128,), jnp.int32),
            pltpu.VMEM((128, _E), jnp.float32),
            pltpu.VMEM((64,), jnp.int32),
            pltpu.VMEM((64, _E), jnp.float32),
            pltpu.SemaphoreType.DMA,
        ],
    )
    def k(wt, qi, et, mf, qe_out, me_out, qi_v, qrows_v, mi_v, mrows_v, sem):
        wid = lax.axis_index("s") * 2 + lax.axis_index("c")
        ca = pltpu.async_copy(qi.at[wid], qi_v, sem)
        cb = pltpu.async_copy(mf.at[pl.ds(wid * 64, 64)], mi_v, sem)
        ca.wait()
        cb.wait()
        cc = pltpu.async_copy(wt.at[qi_v], qrows_v, sem)
        cd = pltpu.async_copy(et.at[mi_v], mrows_v, sem)
        cc.wait()
        cd.wait()
        ce = pltpu.async_copy(qrows_v.at[pl.ds(0, 104)],
                              qe_out.at[pl.ds(wid * 104, 104)], sem)
        cf = pltpu.async_copy(mrows_v, me_out.at[pl.ds(wid * 64, 64)], sem)
        ce.wait()
        cf.wait()

    return k(word_table, qidx_pad, entity_table, mem_flat)


def _sc_walk_gather(entity_table, mids_flat, entities):
    @functools.partial(
        pl.kernel,
        out_type=(
            jax.ShapeDtypeStruct((_B, _E), jnp.float32),
            jax.ShapeDtypeStruct((_B * _M, _E), jnp.float32),
        ),
        mesh=_sc_mesh(),
        scratch_types=[
            pltpu.VMEM((_B,), jnp.int32),
            pltpu.VMEM((64,), jnp.int32),
            pltpu.VMEM((64, _E), jnp.float32),
            pltpu.VMEM((_B, _E), jnp.float32),
            pltpu.SemaphoreType.DMA,
            pltpu.SemaphoreType.DMA,
        ],
    )
    def k(et, mf, ents, eemb_out, memb_out, ents_v, mids_v, erows_v,
          eerows_v, sem, sem2):
        wid = lax.axis_index("s") * 2 + lax.axis_index("c")
        @pl.when(wid < 8)
        def _():
            pltpu.sync_copy(ents, ents_v)
            pltpu.async_copy(et.at[ents_v], eerows_v, sem2)

        pltpu.sync_copy(mf.at[pl.ds(wid * 64, 64)], mids_v)
        pltpu.async_copy(et.at[mids_v], erows_v, sem).wait()
        pltpu.sync_copy(erows_v, memb_out.at[pl.ds(wid * 64, 64)])

        @pl.when(wid < 8)
        def _():
            pltpu.make_async_copy(et.at[ents_v], eerows_v, sem2).wait()
            pltpu.sync_copy(eerows_v.at[pl.ds(wid * 8, 8)],
                            eemb_out.at[pl.ds(wid * 8, 8)])

    return k(entity_table, mids_flat, entities)


def _front(aq, me, watt, wca, wcb, bc, ws):
    t = lax.dot_general(aq, watt, (((1,), (1,)), ((), ())),
                        preferred_element_type=jnp.float32)
    scores = jnp.sum(me * t[:, None, :], axis=2)
    scores = scores - jnp.max(scores, axis=1, keepdims=True)
    ex = jnp.exp(scores)
    w = ex / jnp.sum(ex, axis=1, keepdims=True)
    mv = jnp.sum(w[:, :, None] * me, axis=1)
    comb = jnp.tanh(
        jnp.dot(aq, wca, preferred_element_type=jnp.float32)
        + jnp.dot(mv, wcb, preferred_element_type=jnp.float32)
        + bc)
    s = jnp.dot(comb, ws, preferred_element_type=jnp.float32)
    return comb, s


def _select_tail(i, s_scr, et_ref, key_words, bv, bi, ent_out):
    lg = lax.dot_general(s_scr[...], et_ref[...], (((1,), (1,)), ((), ())),
                         preferred_element_type=jnp.float32) + _gumbel_tile(key_words, i)
    iota_v = lax.broadcasted_iota(jnp.int32, (_B, _VT), 1)
    lg = jnp.where(iota_v + i * _VT < _V, lg, -jnp.inf)
    mval = jnp.max(lg, axis=1, keepdims=True)
    idx = jnp.min(jnp.where(lg == mval, iota_v, _VT), axis=1, keepdims=True)
    gidx = idx + i * _VT
    upd = mval > bv[...]
    bi[...] = jnp.where(upd, gidx, bi[...])
    bv[...] = jnp.where(upd, mval, bv[...])

    @pl.when(i == _NV - 1)
    def _():
        ent_out[...] = bi[...]


def _phase0_call(q_emb3, len_f, W_enc, b_enc2, W_att, W_comb, b_comb2,
                 W_score, mem_emb3, entity_table, key_words):
    def body(q_ref, len_ref, wenc, benc, watt, wc, bc, ws, me, et_ref,
             comb_out, ent_out, s_scr, bv, bi):
        i = pl.program_id(0)

        @pl.when(i == 0)
        def _():
            iota_s = lax.broadcasted_iota(jnp.int32, (_B, _S), 1).astype(jnp.float32)
            mask = (iota_s < len_ref[...]).astype(jnp.float32)
            qp = jnp.sum(q_ref[...] * mask[:, :, None], axis=1) / len_ref[...]
            aq = jnp.tanh(jnp.dot(qp, wenc[...],
                                  preferred_element_type=jnp.float32) + benc[...])
            comb, s = _front(aq, me[...], watt[...], wc[0:_H2, :],
                             wc[_H2:, :], bc[...], ws[...])
            comb_out[...] = comb
            s_scr[...] = s
            bv[...] = jnp.full((_B, 1), -jnp.inf, jnp.float32)
            bi[...] = jnp.zeros((_B, 1), jnp.int32)

        _select_tail(i, s_scr, et_ref, key_words, bv, bi, ent_out)

    full = lambda shape: pl.BlockSpec(shape, lambda i: (0,) * len(shape))
    return pl.pallas_call(
        body,
        grid=(_NV,),
        in_specs=[
            full((_B, _S, _E)),
            full((_B, 1)),
            full((_E, _H2)),
            full((1, _H2)),
            full((_E, _H2)),
            full((_H2 + _E, _H2)),
            full((1, _H2)),
            full((_H2, _E)),
            full((_B, _M, _E)),
            pl.BlockSpec((_VT, _E), lambda i: (i, 0)),
        ],
        out_specs=[full((_B, _H2)), full((_B, 1))],
        out_shape=[
            jax.ShapeDtypeStruct((_B, _H2), jnp.float32),
            jax.ShapeDtypeStruct((_B, 1), jnp.int32),
        ],
        scratch_shapes=[
            pltpu.VMEM((_B, _E), jnp.float32),
            pltpu.VMEM((_B, 1), jnp.float32),
            pltpu.VMEM((_B, 1), jnp.int32),
        ],
        compiler_params=pltpu.CompilerParams(
            dimension_semantics=("arbitrary",)),
    )(q_emb3, len_f, W_enc, b_enc2, W_att, W_comb, b_comb2,
      W_score, mem_emb3, entity_table)


def _phase_walk_call(comb_prev, eemb, W_proj, b_proj2, W_att,
                     W_comb, b_comb2, W_score, mem_emb3,
                     entity_table, key_words):
    def body(cp, ee, wp, bp, watt, wc, bc, ws, me, et_ref,
             comb_out, ent_out, s_scr, bv, bi):
        i = pl.program_id(0)

        @pl.when(i == 0)
        def _():
            aq = jnp.tanh(
                jnp.dot(cp[...], wp[0:_H2, :], preferred_element_type=jnp.float32)
                + jnp.dot(ee[...], wp[_H2:, :], preferred_element_type=jnp.float32)
                + bp[...])
            comb, s = _front(aq, me[...], watt[...], wc[0:_H2, :],
                             wc[_H2:, :], bc[...], ws[...])
            comb_out[...] = comb
            s_scr[...] = s
            bv[...] = jnp.full((_B, 1), -jnp.inf, jnp.float32)
            bi[...] = jnp.zeros((_B, 1), jnp.int32)

        _select_tail(i, s_scr, et_ref, key_words, bv, bi, ent_out)

    full = lambda shape: pl.BlockSpec(shape, lambda i: (0,) * len(shape))
    return pl.pallas_call(
        body,
        grid=(_NV,),
        in_specs=[
            full((_B, _H2)),
            full((_B, _E)),
            full((_H2 + _E, _H2)),
            full((1, _H2)),
            full((_E, _H2)),
            full((_H2 + _E, _H2)),
            full((1, _H2)),
            full((_H2, _E)),
            full((_B, _M, _E)),
            pl.BlockSpec((_VT, _E), lambda i: (i, 0)),
        ],
        out_specs=[full((_B, _H2)), full((_B, 1))],
        out_shape=[
            jax.ShapeDtypeStruct((_B, _H2), jnp.float32),
            jax.ShapeDtypeStruct((_B, 1), jnp.int32),
        ],
        scratch_shapes=[
            pltpu.VMEM((_B, _E), jnp.float32),
            pltpu.VMEM((_B, 1), jnp.float32),
            pltpu.VMEM((_B, 1), jnp.int32),
        ],
        compiler_params=pltpu.CompilerParams(
            dimension_semantics=("arbitrary",)),
    )(comb_prev, eemb, W_proj, b_proj2, W_att, W_comb,
      b_comb2, W_score, mem_emb3, entity_table)


def _phase_final_call(comb_prev, eemb, W_proj, b_proj2, W_att,
                      W_comb, b_comb2, W_score, mem_emb3,
                      entity_table):
    def body(cp, ee, wp, bp, watt, wc, bc, ws, me, et_ref,
             out_ref, s_scr):
        i = pl.program_id(0)

        @pl.when(i == 0)
        def _():
            aq = jnp.tanh(
                jnp.dot(cp[...], wp[0:_H2, :], preferred_element_type=jnp.float32)
                + jnp.dot(ee[...], wp[_H2:, :], preferred_element_type=jnp.float32)
                + bp[...])
            _, s = _front(aq, me[...], watt[...], wc[0:_H2, :],
                          wc[_H2:, :], bc[...], ws[...])
            s_scr[...] = s

        out_ref[...] = lax.dot_general(
            s_scr[...], et_ref[...], (((1,), (1,)), ((), ())),
            preferred_element_type=jnp.float32)

    full = lambda shape: pl.BlockSpec(shape, lambda i: (0,) * len(shape))
    return pl.pallas_call(
        body,
        grid=(_NV,),
        in_specs=[
            full((_B, _H2)),
            full((_B, _E)),
            full((_H2 + _E, _H2)),
            full((1, _H2)),
            full((_E, _H2)),
            full((_H2 + _E, _H2)),
            full((1, _H2)),
            full((_H2, _E)),
            full((_B, _M, _E)),
            pl.BlockSpec((_VT, _E), lambda i: (i, 0)),
        ],
        out_specs=pl.BlockSpec((_B, _VT), lambda i: (0, i)),
        out_shape=jax.ShapeDtypeStruct((_B, _V), jnp.float32),
        scratch_shapes=[pltpu.VMEM((_B, _E), jnp.float32)],
        compiler_params=pltpu.CompilerParams(
            dimension_semantics=("arbitrary",)),
    )(comb_prev, eemb, W_proj, b_proj2, W_att, W_comb,
      b_comb2, W_score, mem_emb3, entity_table)


def kernel(memory, question, question_lengths, memory_graph, entity_table,
           word_table, W_enc, b_enc, W_att, W_comb, b_comb, W_score, W_proj,
           b_proj):
    memory = memory.astype(jnp.int32)
    question = question.astype(jnp.int32)
    memory_graph = memory_graph.astype(jnp.int32)

    len_f = jnp.maximum(question_lengths, 1).astype(jnp.float32).reshape(_B, 1)
    b_enc2 = b_enc.reshape(1, _H2)
    b_comb2 = b_comb.reshape(1, _H2)
    b_proj2 = b_proj.reshape(1, _H2)

    qflat = jnp.pad(question.reshape(_B * _S), (0, _NW * 104 - _B * _S))
    qidx_pad = jnp.pad(qflat.reshape(_NW, 104), ((0, 0), (0, 24)))
    mem_flat = memory.reshape(_B * _M)

    q_emb, mem_emb0 = _sc_initial_gather(word_table, qidx_pad, entity_table,
                                         mem_flat)
    comb0, ent0 = _phase0_call(
        q_emb[:_B * _S].reshape(_B, _S, _E), len_f, W_enc, b_enc2, W_att,
        W_comb, b_comb2, W_score, mem_emb0.reshape(_B, _M, _E),
        entity_table, _K0)

    mids1 = jnp.take(memory_graph, ent0.reshape(_B), axis=0).reshape(_B * _M)
    eemb1, mem_emb1 = _sc_walk_gather(entity_table, mids1, ent0.reshape(_B))
    comb1, ent1 = _phase_walk_call(
        comb0, eemb1, W_proj, b_proj2, W_att, W_comb,
        b_comb2, W_score, mem_emb1.reshape(_B, _M, _E), entity_table, _K1)

    mids2 = jnp.take(memory_graph, ent1.reshape(_B), axis=0).reshape(_B * _M)
    eemb2, mem_emb2 = _sc_walk_gather(entity_table, mids2, ent1.reshape(_B))
    logits = _phase_final_call(
        comb1, eemb2, W_proj, b_proj2, W_att, W_comb,
        b_comb2, W_score, mem_emb2.reshape(_B, _M, _E), entity_table)
    return logits


# speedup vs baseline: 1.5799x; 1.0844x over previous
import functools

import jax
import jax.numpy as jnp
from jax import lax
from jax.experimental import pallas as pl
from jax.experimental.pallas import tpu as pltpu
from jax.experimental.pallas import tpu_sc as plsc

_V = 100000
_E = 128
_H2 = 1024
_B = 64
_S = 50
_M = 32
_VT = 5120
_NV = (_V + _VT - 1) // _VT
_NW = 32


_TF_R1 = (13, 15, 26, 6)
_TF_R2 = (17, 29, 16, 24)
_TF_PARITY = 0x1BD11BDA


def _tf_py(k1, k2, x0, x1):
    msk = 0xFFFFFFFF
    rot = lambda x, r: ((x << r) | (x >> (32 - r))) & msk
    ks = [k1, k2, (k1 ^ k2 ^ _TF_PARITY) & msk]
    x0 = (x0 + ks[0]) & msk
    x1 = (x1 + ks[1]) & msk
    for j, rs in enumerate((_TF_R1, _TF_R2, _TF_R1, _TF_R2, _TF_R1)):
        for r in rs:
            x0 = (x0 + x1) & msk
            x1 = x0 ^ rot(x1, r)
        x0 = (x0 + ks[(j + 1) % 3]) & msk
        x1 = (x1 + ks[(j + 2) % 3] + j + 1) & msk
    return x0, x1


def _fold_in_py(i):
    return _tf_py(0, 42, 0, i)


def _i32c(x):
    x &= 0xFFFFFFFF
    return x - (1 << 32) if x >= (1 << 31) else x


def _tf_bits(k1, k2, p):
    c = lambda v: jnp.int32(_i32c(v))
    ks0, ks1 = k1, k2
    ks2 = k1 ^ k2 ^ _TF_PARITY

    def rot(x, r):
        return lax.shift_left(x, jnp.int32(r)) | lax.shift_right_logical(
            x, jnp.int32(32 - r))

    x0 = jnp.full(p.shape, c(ks0), jnp.int32)
    x1 = p + c(ks1)
    ks = (ks0, ks1, ks2)
    for j, rs in enumerate((_TF_R1, _TF_R2, _TF_R1, _TF_R2, _TF_R1)):
        for r in rs:
            x0 = x0 + x1
            x1 = x0 ^ rot(x1, r)
        x0 = x0 + c(ks[(j + 1) % 3])
        x1 = x1 + c(ks[(j + 2) % 3] + j + 1)
    return x0 ^ x1


_K0 = _fold_in_py(0)
_K1 = _fold_in_py(1)
_F32_TINY = 1.1754943508222875e-38


def _gumbel_tile(key_words, i):
    row = lax.broadcasted_iota(jnp.int32, (_B, _VT), 0)
    col = lax.broadcasted_iota(jnp.int32, (_B, _VT), 1) + i * _VT
    p = row * _V + col
    bits = _tf_bits(key_words[0], key_words[1], p)
    fb = lax.shift_right_logical(bits, jnp.int32(9)) | jnp.int32(0x3F800000)
    f = lax.bitcast_convert_type(fb, jnp.float32) - jnp.float32(1.0)
    u = jnp.maximum(f, jnp.float32(_F32_TINY))
    return -jnp.log(-jnp.log(u))


def _sc_mesh():
    return plsc.VectorSubcoreMesh(core_axis_name="c", subcore_axis_name="s")


def _sc_initial_gather(word_table, qidx_flat, entity_table, mem_flat):
    @functools.partial(
        pl.kernel,
        out_type=(
            jax.ShapeDtypeStruct((_NW * 104, _E), jnp.float32),
            jax.ShapeDtypeStruct((_B * _M, _E), jnp.float32),
        ),
        mesh=_sc_mesh(),
        scratch_types=[
            pltpu.VMEM((104,), jnp.int32),
            pltpu.VMEM((104, _E), jnp.float32),
            pltpu.VMEM((64,), jnp.int32),
            pltpu.VMEM((64, _E), jnp.float32),
            pltpu.SemaphoreType.DMA,
            pltpu.SemaphoreType.DMA,
        ],
    )
    def k(wt, qi, et, mf, qe_out, me_out, qi_v, qrows_v, mi_v, mrows_v, sem, sem2):
        wid = lax.axis_index("s") * 2 + lax.axis_index("c")
        pltpu.sync_copy(mf.at[pl.ds(wid * 64, 64)], mi_v)
        pltpu.async_copy(et.at[mi_v], mrows_v, sem2)
        pltpu.sync_copy(qi.at[pl.ds(wid * 128, 104)], qi_v)
        pltpu.async_copy(wt.at[qi_v], qrows_v, sem).wait()
        pltpu.sync_copy(qrows_v, qe_out.at[pl.ds(wid * 104, 104)])
        pltpu.make_async_copy(et.at[mi_v], mrows_v, sem2).wait()
        pltpu.sync_copy(mrows_v, me_out.at[pl.ds(wid * 64, 64)])

    return k(word_table, qidx_flat, entity_table, mem_flat)


def _sc_walk_gather(entity_table, mids_flat, entities):
    @functools.partial(
        pl.kernel,
        out_type=(
            jax.ShapeDtypeStruct((_B, _E), jnp.float32),
            jax.ShapeDtypeStruct((_B * _M, _E), jnp.float32),
        ),
        mesh=_sc_mesh(),
        scratch_types=[
            pltpu.VMEM((_B,), jnp.int32),
            pltpu.VMEM((64,), jnp.int32),
            pltpu.VMEM((64, _E), jnp.float32),
            pltpu.VMEM((_B, _E), jnp.float32),
            pltpu.SemaphoreType.DMA,
            pltpu.SemaphoreType.DMA,
        ],
    )
    def k(et, mf, ents, eemb_out, memb_out, ents_v, mids_v, erows_v,
          eerows_v, sem, sem2):
        wid = lax.axis_index("s") * 2 + lax.axis_index("c")
        @pl.when(wid < 8)
        def _():
            pltpu.sync_copy(ents, ents_v)
            pltpu.async_copy(et.at[ents_v], eerows_v, sem2)

        pltpu.sync_copy(mf.at[pl.ds(wid * 64, 64)], mids_v)
        pltpu.async_copy(et.at[mids_v], erows_v, sem).wait()
        pltpu.sync_copy(erows_v, memb_out.at[pl.ds(wid * 64, 64)])

        @pl.when(wid < 8)
        def _():
            pltpu.make_async_copy(et.at[ents_v], eerows_v, sem2).wait()
            pltpu.sync_copy(eerows_v.at[pl.ds(wid * 8, 8)],
                            eemb_out.at[pl.ds(wid * 8, 8)])

    return k(entity_table, mids_flat, entities)


def _front(aq, me, watt, wca, wcb, bc, ws):
    t = lax.dot_general(aq, watt, (((1,), (1,)), ((), ())),
                        preferred_element_type=jnp.float32)
    scores = jnp.sum(me * t[:, None, :], axis=2)
    scores = scores - jnp.max(scores, axis=1, keepdims=True)
    ex = jnp.exp(scores)
    w = ex / jnp.sum(ex, axis=1, keepdims=True)
    mv = jnp.sum(w[:, :, None] * me, axis=1)
    comb = jnp.tanh(
        jnp.dot(aq, wca, preferred_element_type=jnp.float32)
        + jnp.dot(mv, wcb, preferred_element_type=jnp.float32)
        + bc)
    s = jnp.dot(comb, ws, preferred_element_type=jnp.float32)
    return comb, s


def _select_tail(i, s_scr, et_ref, key_words, bv, bi, ent_out):
    lg = lax.dot_general(s_scr[...], et_ref[...], (((1,), (1,)), ((), ())),
                         preferred_element_type=jnp.float32) + _gumbel_tile(key_words, i)
    iota_v = lax.broadcasted_iota(jnp.int32, (_B, _VT), 1)
    lg = jnp.where(iota_v + i * _VT < _V, lg, -jnp.inf)
    mval = jnp.max(lg, axis=1, keepdims=True)
    idx = jnp.min(jnp.where(lg == mval, iota_v, _VT), axis=1, keepdims=True)
    gidx = idx + i * _VT
    upd = mval > bv[...]
    bi[...] = jnp.where(upd, gidx, bi[...])
    bv[...] = jnp.where(upd, mval, bv[...])

    @pl.when(i == _NV - 1)
    def _():
        ent_out[...] = bi[...]


def _phase0_call(q_emb3, len_f, W_enc, b_enc2, W_att, W_comb, b_comb2,
                 W_score, mem_emb3, entity_table, key_words):
    def body(q_ref, len_ref, wenc, benc, watt, wc, bc, ws, me, et_ref,
             comb_out, ent_out, s_scr, bv, bi):
        i = pl.program_id(0)

        @pl.when(i == 0)
        def _():
            iota_s = lax.broadcasted_iota(jnp.int32, (_B, _S), 1).astype(jnp.float32)
            mask = (iota_s < len_ref[...]).astype(jnp.float32)
            qp = jnp.sum(q_ref[...] * mask[:, :, None], axis=1) / len_ref[...]
            aq = jnp.tanh(jnp.dot(qp, wenc[...],
                                  preferred_element_type=jnp.float32) + benc[...])
            comb, s = _front(aq, me[...], watt[...], wc[0:_H2, :],
                             wc[_H2:, :], bc[...], ws[...])
            comb_out[...] = comb
            s_scr[...] = s
            bv[...] = jnp.full((_B, 1), -jnp.inf, jnp.float32)
            bi[...] = jnp.zeros((_B, 1), jnp.int32)

        _select_tail(i, s_scr, et_ref, key_words, bv, bi, ent_out)

    full = lambda shape: pl.BlockSpec(shape, lambda i: (0,) * len(shape))
    return pl.pallas_call(
        body,
        grid=(_NV,),
        in_specs=[
            full((_B, _S, _E)),
            full((_B, 1)),
            full((_E, _H2)),
            full((1, _H2)),
            full((_E, _H2)),
            full((_H2 + _E, _H2)),
            full((1, _H2)),
            full((_H2, _E)),
            full((_B, _M, _E)),
            pl.BlockSpec((_VT, _E), lambda i: (i, 0)),
        ],
        out_specs=[full((_B, _H2)), full((_B, 1))],
        out_shape=[
            jax.ShapeDtypeStruct((_B, _H2), jnp.float32),
            jax.ShapeDtypeStruct((_B, 1), jnp.int32),
        ],
        scratch_shapes=[
            pltpu.VMEM((_B, _E), jnp.float32),
            pltpu.VMEM((_B, 1), jnp.float32),
            pltpu.VMEM((_B, 1), jnp.int32),
        ],
        compiler_params=pltpu.CompilerParams(
            dimension_semantics=("arbitrary",)),
    )(q_emb3, len_f, W_enc, b_enc2, W_att, W_comb, b_comb2,
      W_score, mem_emb3, entity_table)


def _phase_walk_call(comb_prev, eemb, W_proj, b_proj2, W_att,
                     W_comb, b_comb2, W_score, mem_emb3,
                     entity_table, key_words):
    def body(cp, ee, wp, bp, watt, wc, bc, ws, me, et_ref,
             comb_out, ent_out, s_scr, bv, bi):
        i = pl.program_id(0)

        @pl.when(i == 0)
        def _():
            aq = jnp.tanh(
                jnp.dot(cp[...], wp[0:_H2, :], preferred_element_type=jnp.float32)
                + jnp.dot(ee[...], wp[_H2:, :], preferred_element_type=jnp.float32)
                + bp[...])
            comb, s = _front(aq, me[...], watt[...], wc[0:_H2, :],
                             wc[_H2:, :], bc[...], ws[...])
            comb_out[...] = comb
            s_scr[...] = s
            bv[...] = jnp.full((_B, 1), -jnp.inf, jnp.float32)
            bi[...] = jnp.zeros((_B, 1), jnp.int32)

        _select_tail(i, s_scr, et_ref, key_words, bv, bi, ent_out)

    full = lambda shape: pl.BlockSpec(shape, lambda i: (0,) * len(shape))
    return pl.pallas_call(
        body,
        grid=(_NV,),
        in_specs=[
            full((_B, _H2)),
            full((_B, _E)),
            full((_H2 + _E, _H2)),
            full((1, _H2)),
            full((_E, _H2)),
            full((_H2 + _E, _H2)),
            full((1, _H2)),
            full((_H2, _E)),
            full((_B, _M, _E)),
            pl.BlockSpec((_VT, _E), lambda i: (i, 0)),
        ],
        out_specs=[full((_B, _H2)), full((_B, 1))],
        out_shape=[
            jax.ShapeDtypeStruct((_B, _H2), jnp.float32),
            jax.ShapeDtypeStruct((_B, 1), jnp.int32),
        ],
        scratch_shapes=[
            pltpu.VMEM((_B, _E), jnp.float32),
            pltpu.VMEM((_B, 1), jnp.float32),
            pltpu.VMEM((_B, 1), jnp.int32),
        ],
        compiler_params=pltpu.CompilerParams(
            dimension_semantics=("arbitrary",)),
    )(comb_prev, eemb, W_proj, b_proj2, W_att, W_comb,
      b_comb2, W_score, mem_emb3, entity_table)


def _phase_final_call(comb_prev, eemb, W_proj, b_proj2, W_att,
                      W_comb, b_comb2, W_score, mem_emb3,
                      entity_table):
    def body(cp, ee, wp, bp, watt, wc, bc, ws, me, et_ref,
             out_ref, s_scr):
        i = pl.program_id(0)

        @pl.when(i == 0)
        def _():
            aq = jnp.tanh(
                jnp.dot(cp[...], wp[0:_H2, :], preferred_element_type=jnp.float32)
                + jnp.dot(ee[...], wp[_H2:, :], preferred_element_type=jnp.float32)
                + bp[...])
            _, s = _front(aq, me[...], watt[...], wc[0:_H2, :],
                          wc[_H2:, :], bc[...], ws[...])
            s_scr[...] = s

        out_ref[...] = lax.dot_general(
            s_scr[...], et_ref[...], (((1,), (1,)), ((), ())),
            preferred_element_type=jnp.float32)

    full = lambda shape: pl.BlockSpec(shape, lambda i: (0,) * len(shape))
    return pl.pallas_call(
        body,
        grid=(_NV,),
        in_specs=[
            full((_B, _H2)),
            full((_B, _E)),
            full((_H2 + _E, _H2)),
            full((1, _H2)),
            full((_E, _H2)),
            full((_H2 + _E, _H2)),
            full((1, _H2)),
            full((_H2, _E)),
            full((_B, _M, _E)),
            pl.BlockSpec((_VT, _E), lambda i: (i, 0)),
        ],
        out_specs=pl.BlockSpec((_B, _VT), lambda i: (0, i)),
        out_shape=jax.ShapeDtypeStruct((_B, _V), jnp.float32),
        scratch_shapes=[pltpu.VMEM((_B, _E), jnp.float32)],
        compiler_params=pltpu.CompilerParams(
            dimension_semantics=("arbitrary",)),
    )(comb_prev, eemb, W_proj, b_proj2, W_att, W_comb,
      b_comb2, W_score, mem_emb3, entity_table)


def kernel(memory, question, question_lengths, memory_graph, entity_table,
           word_table, W_enc, b_enc, W_att, W_comb, b_comb, W_score, W_proj,
           b_proj):
    memory = memory.astype(jnp.int32)
    question = question.astype(jnp.int32)
    memory_graph = memory_graph.astype(jnp.int32)

    len_f = jnp.maximum(question_lengths, 1).astype(jnp.float32).reshape(_B, 1)
    b_enc2 = b_enc.reshape(1, _H2)
    b_comb2 = b_comb.reshape(1, _H2)
    b_proj2 = b_proj.reshape(1, _H2)

    qflat = jnp.pad(question.reshape(_B * _S), (0, _NW * 104 - _B * _S))
    qidx_flat = jnp.pad(qflat.reshape(_NW, 104), ((0, 0), (0, 24))).reshape(_NW * 128)
    mem_flat = memory.reshape(_B * _M)

    q_emb, mem_emb0 = _sc_initial_gather(word_table, qidx_flat, entity_table,
                                         mem_flat)
    comb0, ent0 = _phase0_call(
        q_emb[:_B * _S].reshape(_B, _S, _E), len_f, W_enc, b_enc2, W_att,
        W_comb, b_comb2, W_score, mem_emb0.reshape(_B, _M, _E),
        entity_table, _K0)

    mids1 = jnp.take(memory_graph, ent0.reshape(_B), axis=0).reshape(_B * _M)
    eemb1, mem_emb1 = _sc_walk_gather(entity_table, mids1, ent0.reshape(_B))
    comb1, ent1 = _phase_walk_call(
        comb0, eemb1, W_proj, b_proj2, W_att, W_comb,
        b_comb2, W_score, mem_emb1.reshape(_B, _M, _E), entity_table, _K1)

    mids2 = jnp.take(memory_graph, ent1.reshape(_B), axis=0).reshape(_B * _M)
    eemb2, mem_emb2 = _sc_walk_gather(entity_table, mids2, ent1.reshape(_B))
    logits = _phase_final_call(
        comb1, eemb2, W_proj, b_proj2, W_att, W_comb,
        b_comb2, W_score, mem_emb2.reshape(_B, _M, _E), entity_table)
    return logits


# speedup vs baseline: 1.6231x; 1.0273x over previous
import functools

import jax
import jax.numpy as jnp
from jax import lax
from jax.experimental import pallas as pl
from jax.experimental.pallas import tpu as pltpu
from jax.experimental.pallas import tpu_sc as plsc

_V = 100000
_E = 128
_H2 = 1024
_B = 64
_S = 50
_M = 32
_VT = 5120
_NV = (_V + _VT - 1) // _VT
_NW = 32


_TF_R1 = (13, 15, 26, 6)
_TF_R2 = (17, 29, 16, 24)
_TF_PARITY = 0x1BD11BDA


def _tf_py(k1, k2, x0, x1):
    msk = 0xFFFFFFFF
    rot = lambda x, r: ((x << r) | (x >> (32 - r))) & msk
    ks = [k1, k2, (k1 ^ k2 ^ _TF_PARITY) & msk]
    x0 = (x0 + ks[0]) & msk
    x1 = (x1 + ks[1]) & msk
    for j, rs in enumerate((_TF_R1, _TF_R2, _TF_R1, _TF_R2, _TF_R1)):
        for r in rs:
            x0 = (x0 + x1) & msk
            x1 = x0 ^ rot(x1, r)
        x0 = (x0 + ks[(j + 1) % 3]) & msk
        x1 = (x1 + ks[(j + 2) % 3] + j + 1) & msk
    return x0, x1


def _fold_in_py(i):
    return _tf_py(0, 42, 0, i)


def _i32c(x):
    x &= 0xFFFFFFFF
    return x - (1 << 32) if x >= (1 << 31) else x


def _tf_bits(k1, k2, p):
    c = lambda v: jnp.int32(_i32c(v))
    ks0, ks1 = k1, k2
    ks2 = k1 ^ k2 ^ _TF_PARITY

    def rot(x, r):
        return lax.shift_left(x, jnp.int32(r)) | lax.shift_right_logical(
            x, jnp.int32(32 - r))

    x0 = jnp.full(p.shape, c(ks0), jnp.int32)
    x1 = p + c(ks1)
    ks = (ks0, ks1, ks2)
    for j, rs in enumerate((_TF_R1, _TF_R2, _TF_R1, _TF_R2, _TF_R1)):
        for r in rs:
            x0 = x0 + x1
            x1 = x0 ^ rot(x1, r)
        x0 = x0 + c(ks[(j + 1) % 3])
        x1 = x1 + c(ks[(j + 2) % 3] + j + 1)
    return x0 ^ x1


_K0 = _fold_in_py(0)
_K1 = _fold_in_py(1)
_F32_TINY = 1.1754943508222875e-38


def _gumbel_tile(key_words, i):
    row = lax.broadcasted_iota(jnp.int32, (_B, _VT), 0)
    col = lax.broadcasted_iota(jnp.int32, (_B, _VT), 1) + i * _VT
    p = row * _V + col
    bits = _tf_bits(key_words[0], key_words[1], p)
    fb = lax.shift_right_logical(bits, jnp.int32(9)) | jnp.int32(0x3F800000)
    f = lax.bitcast_convert_type(fb, jnp.float32) - jnp.float32(1.0)
    u = jnp.maximum(f, jnp.float32(_F32_TINY))
    return -jnp.log(-jnp.log(u))


def _sc_mesh():
    return plsc.VectorSubcoreMesh(core_axis_name="c", subcore_axis_name="s")


def _sc_initial_gather(word_table, qidx_flat, entity_table, mem_flat):
    @functools.partial(
        pl.kernel,
        out_type=(
            jax.ShapeDtypeStruct((_NW * 104, _E), jnp.float32),
            jax.ShapeDtypeStruct((_B * _M, _E), jnp.float32),
        ),
        mesh=_sc_mesh(),
        scratch_types=[
            pltpu.VMEM((104,), jnp.int32),
            pltpu.VMEM((104, _E), jnp.float32),
            pltpu.VMEM((64,), jnp.int32),
            pltpu.VMEM((64, _E), jnp.float32),
            pltpu.SemaphoreType.DMA,
            pltpu.SemaphoreType.DMA,
        ],
    )
    def k(wt, qi, et, mf, qe_out, me_out, qi_v, qrows_v, mi_v, mrows_v, sem, sem2):
        wid = lax.axis_index("s") * 2 + lax.axis_index("c")
        pltpu.sync_copy(mf.at[pl.ds(wid * 64, 64)], mi_v)
        pltpu.async_copy(et.at[mi_v], mrows_v, sem2)
        pltpu.sync_copy(qi.at[pl.ds(wid * 128, 104)], qi_v)
        pltpu.async_copy(wt.at[qi_v], qrows_v, sem).wait()
        pltpu.sync_copy(qrows_v, qe_out.at[pl.ds(wid * 104, 104)])
        pltpu.make_async_copy(et.at[mi_v], mrows_v, sem2).wait()
        pltpu.sync_copy(mrows_v, me_out.at[pl.ds(wid * 64, 64)])

    return k(word_table, qidx_flat, entity_table, mem_flat)


def _sc_walk_gather(entity_table, memory_graph, entities):
    @functools.partial(
        pl.kernel,
        out_type=(
            jax.ShapeDtypeStruct((_B, _E), jnp.float32),
            jax.ShapeDtypeStruct((_B * _M, _E), jnp.float32),
        ),
        mesh=_sc_mesh(),
        scratch_types=[
            pltpu.VMEM((_B + 16,), jnp.int32),
            pltpu.VMEM((8, _M), jnp.int32),
            pltpu.VMEM((8, _M), jnp.int32),
            pltpu.VMEM((64,), jnp.int32),
            pltpu.VMEM((64, _E), jnp.float32),
            pltpu.VMEM((_B, _E), jnp.float32),
            pltpu.SemaphoreType.DMA,
            pltpu.SemaphoreType.DMA,
        ],
    )
    def k(et, mg, ents, eemb_out, memb_out, ents_v, blk0_v, blk1_v, idx_v,
          erows_v, eerows_v, sem, sem2):
        wid = lax.axis_index("s") * 2 + lax.axis_index("c")
        b0 = wid * 2
        pltpu.sync_copy(ents, ents_v.at[pl.ds(0, _B)])

        @pl.when(wid < 8)
        def _():
            pltpu.async_copy(et.at[ents_v.at[pl.ds(0, _B)]], eerows_v, sem2)

        e0 = ents_v[pl.ds(b0, 16)][0]
        e1 = ents_v[pl.ds(b0 + 1, 16)][0]
        ca = pltpu.async_copy(
            mg.at[pl.ds(pl.multiple_of((e0 // 8) * 8, 8), 8)], blk0_v, sem)
        cb = pltpu.async_copy(
            mg.at[pl.ds(pl.multiple_of((e1 // 8) * 8, 8), 8)], blk1_v, sem)
        ca.wait()
        cb.wait()
        for r, (blk, e) in enumerate(((blk0_v, e0), (blk1_v, e1))):
            for c in range(2):
                idx_v[pl.ds((r * 2 + c) * 16, 16)] = blk[e % 8, pl.ds(c * 16, 16)]
        pltpu.async_copy(et.at[idx_v], erows_v, sem).wait()
        pltpu.sync_copy(erows_v, memb_out.at[pl.ds(b0 * _M, 64)])

        @pl.when(wid < 8)
        def _():
            pltpu.make_async_copy(et.at[ents_v.at[pl.ds(0, _B)]], eerows_v,
                                  sem2).wait()
            pltpu.sync_copy(eerows_v.at[pl.ds(wid * 8, 8)],
                            eemb_out.at[pl.ds(wid * 8, 8)])

    return k(entity_table, memory_graph, entities)


def _front(aq, me, watt, wca, wcb, bc, ws):
    t = lax.dot_general(aq, watt, (((1,), (1,)), ((), ())),
                        preferred_element_type=jnp.float32)
    scores = jnp.sum(me * t[:, None, :], axis=2)
    scores = scores - jnp.max(scores, axis=1, keepdims=True)
    ex = jnp.exp(scores)
    w = ex / jnp.sum(ex, axis=1, keepdims=True)
    mv = jnp.sum(w[:, :, None] * me, axis=1)
    comb = jnp.tanh(
        jnp.dot(aq, wca, preferred_element_type=jnp.float32)
        + jnp.dot(mv, wcb, preferred_element_type=jnp.float32)
        + bc)
    s = jnp.dot(comb, ws, preferred_element_type=jnp.float32)
    return comb, s


def _select_tail(i, s_scr, et_ref, key_words, bv, bi, ent_out):
    lg = lax.dot_general(s_scr[...], et_ref[...], (((1,), (1,)), ((), ())),
                         preferred_element_type=jnp.float32) + _gumbel_tile(key_words, i)
    iota_v = lax.broadcasted_iota(jnp.int32, (_B, _VT), 1)
    lg = jnp.where(iota_v + i * _VT < _V, lg, -jnp.inf)
    mval = jnp.max(lg, axis=1, keepdims=True)
    idx = jnp.min(jnp.where(lg == mval, iota_v, _VT), axis=1, keepdims=True)
    gidx = idx + i * _VT
    upd = mval > bv[...]
    bi[...] = jnp.where(upd, gidx, bi[...])
    bv[...] = jnp.where(upd, mval, bv[...])

    @pl.when(i == _NV - 1)
    def _():
        ent_out[...] = bi[...]


def _phase0_call(q_emb3, len_f, W_enc, b_enc2, W_att, W_comb, b_comb2,
                 W_score, mem_emb3, entity_table, key_words):
    def body(q_ref, len_ref, wenc, benc, watt, wc, bc, ws, me, et_ref,
             comb_out, ent_out, s_scr, bv, bi):
        i = pl.program_id(0)

        @pl.when(i == 0)
        def _():
            iota_s = lax.broadcasted_iota(jnp.int32, (_B, _S), 1).astype(jnp.float32)
            mask = (iota_s < len_ref[...]).astype(jnp.float32)
            qp = jnp.sum(q_ref[...] * mask[:, :, None], axis=1) / len_ref[...]
            aq = jnp.tanh(jnp.dot(qp, wenc[...],
                                  preferred_element_type=jnp.float32) + benc[...])
            comb, s = _front(aq, me[...], watt[...], wc[0:_H2, :],
                             wc[_H2:, :], bc[...], ws[...])
            comb_out[...] = comb
            s_scr[...] = s
            bv[...] = jnp.full((_B, 1), -jnp.inf, jnp.float32)
            bi[...] = jnp.zeros((_B, 1), jnp.int32)

        _select_tail(i, s_scr, et_ref, key_words, bv, bi, ent_out)

    full = lambda shape: pl.BlockSpec(shape, lambda i: (0,) * len(shape))
    return pl.pallas_call(
        body,
        grid=(_NV,),
        in_specs=[
            full((_B, _S, _E)),
            full((_B, 1)),
            full((_E, _H2)),
            full((1, _H2)),
            full((_E, _H2)),
            full((_H2 + _E, _H2)),
            full((1, _H2)),
            full((_H2, _E)),
            full((_B, _M, _E)),
            pl.BlockSpec((_VT, _E), lambda i: (i, 0)),
        ],
        out_specs=[full((_B, _H2)), full((_B, 1))],
        out_shape=[
            jax.ShapeDtypeStruct((_B, _H2), jnp.float32),
            jax.ShapeDtypeStruct((_B, 1), jnp.int32),
        ],
        scratch_shapes=[
            pltpu.VMEM((_B, _E), jnp.float32),
            pltpu.VMEM((_B, 1), jnp.float32),
            pltpu.VMEM((_B, 1), jnp.int32),
        ],
        compiler_params=pltpu.CompilerParams(
            dimension_semantics=("arbitrary",)),
    )(q_emb3, len_f, W_enc, b_enc2, W_att, W_comb, b_comb2,
      W_score, mem_emb3, entity_table)


def _phase_walk_call(comb_prev, eemb, W_proj, b_proj2, W_att,
                     W_comb, b_comb2, W_score, mem_emb3,
                     entity_table, key_words):
    def body(cp, ee, wp, bp, watt, wc, bc, ws, me, et_ref,
             comb_out, ent_out, s_scr, bv, bi):
        i = pl.program_id(0)

        @pl.when(i == 0)
        def _():
            aq = jnp.tanh(
                jnp.dot(cp[...], wp[0:_H2, :], preferred_element_type=jnp.float32)
                + jnp.dot(ee[...], wp[_H2:, :], preferred_element_type=jnp.float32)
                + bp[...])
            comb, s = _front(aq, me[...], watt[...], wc[0:_H2, :],
                             wc[_H2:, :], bc[...], ws[...])
            comb_out[...] = comb
            s_scr[...] = s
            bv[...] = jnp.full((_B, 1), -jnp.inf, jnp.float32)
            bi[...] = jnp.zeros((_B, 1), jnp.int32)

        _select_tail(i, s_scr, et_ref, key_words, bv, bi, ent_out)

    full = lambda shape: pl.BlockSpec(shape, lambda i: (0,) * len(shape))
    return pl.pallas_call(
        body,
        grid=(_NV,),
        in_specs=[
            full((_B, _H2)),
            full((_B, _E)),
            full((_H2 + _E, _H2)),
            full((1, _H2)),
            full((_E, _H2)),
            full((_H2 + _E, _H2)),
            full((1, _H2)),
            full((_H2, _E)),
            full((_B, _M, _E)),
            pl.BlockSpec((_VT, _E), lambda i: (i, 0)),
        ],
        out_specs=[full((_B, _H2)), full((_B, 1))],
        out_shape=[
            jax.ShapeDtypeStruct((_B, _H2), jnp.float32),
            jax.ShapeDtypeStruct((_B, 1), jnp.int32),
        ],
        scratch_shapes=[
            pltpu.VMEM((_B, _E), jnp.float32),
            pltpu.VMEM((_B, 1), jnp.float32),
            pltpu.VMEM((_B, 1), jnp.int32),
        ],
        compiler_params=pltpu.CompilerParams(
            dimension_semantics=("arbitrary",)),
    )(comb_prev, eemb, W_proj, b_proj2, W_att, W_comb,
      b_comb2, W_score, mem_emb3, entity_table)


def _phase_final_call(comb_prev, eemb, W_proj, b_proj2, W_att,
                      W_comb, b_comb2, W_score, mem_emb3,
                      entity_table):
    def body(cp, ee, wp, bp, watt, wc, bc, ws, me, et_ref,
             out_ref, s_scr):
        i = pl.program_id(0)

        @pl.when(i == 0)
        def _():
            aq = jnp.tanh(
                jnp.dot(cp[...], wp[0:_H2, :], preferred_element_type=jnp.float32)
                + jnp.dot(ee[...], wp[_H2:, :], preferred_element_type=jnp.float32)
                + bp[...])
            _, s = _front(aq, me[...], watt[...], wc[0:_H2, :],
                          wc[_H2:, :], bc[...], ws[...])
            s_scr[...] = s

        out_ref[...] = lax.dot_general(
            s_scr[...], et_ref[...], (((1,), (1,)), ((), ())),
            preferred_element_type=jnp.float32)

    full = lambda shape: pl.BlockSpec(shape, lambda i: (0,) * len(shape))
    return pl.pallas_call(
        body,
        grid=(_NV,),
        in_specs=[
            full((_B, _H2)),
            full((_B, _E)),
            full((_H2 + _E, _H2)),
            full((1, _H2)),
            full((_E, _H2)),
            full((_H2 + _E, _H2)),
            full((1, _H2)),
            full((_H2, _E)),
            full((_B, _M, _E)),
            pl.BlockSpec((_VT, _E), lambda i: (i, 0)),
        ],
        out_specs=pl.BlockSpec((_B, _VT), lambda i: (0, i)),
        out_shape=jax.ShapeDtypeStruct((_B, _V), jnp.float32),
        scratch_shapes=[pltpu.VMEM((_B, _E), jnp.float32)],
        compiler_params=pltpu.CompilerParams(
            dimension_semantics=("arbitrary",)),
    )(comb_prev, eemb, W_proj, b_proj2, W_att, W_comb,
      b_comb2, W_score, mem_emb3, entity_table)


def kernel(memory, question, question_lengths, memory_graph, entity_table,
           word_table, W_enc, b_enc, W_att, W_comb, b_comb, W_score, W_proj,
           b_proj):
    memory = memory.astype(jnp.int32)
    question = question.astype(jnp.int32)
    memory_graph = memory_graph.astype(jnp.int32)

    len_f = jnp.maximum(question_lengths, 1).astype(jnp.float32).reshape(_B, 1)
    b_enc2 = b_enc.reshape(1, _H2)
    b_comb2 = b_comb.reshape(1, _H2)
    b_proj2 = b_proj.reshape(1, _H2)

    qflat = jnp.pad(question.reshape(_B * _S), (0, _NW * 104 - _B * _S))
    qidx_flat = jnp.pad(qflat.reshape(_NW, 104), ((0, 0), (0, 24))).reshape(_NW * 128)
    mem_flat = memory.reshape(_B * _M)

    q_emb, mem_emb0 = _sc_initial_gather(word_table, qidx_flat, entity_table,
                                         mem_flat)
    comb0, ent0 = _phase0_call(
        q_emb[:_B * _S].reshape(_B, _S, _E), len_f, W_enc, b_enc2, W_att,
        W_comb, b_comb2, W_score, mem_emb0.reshape(_B, _M, _E),
        entity_table, _K0)

    eemb1, mem_emb1 = _sc_walk_gather(entity_table, memory_graph, ent0.reshape(_B))
    comb1, ent1 = _phase_walk_call(
        comb0, eemb1, W_proj, b_proj2, W_att, W_comb,
        b_comb2, W_score, mem_emb1.reshape(_B, _M, _E), entity_table, _K1)

    eemb2, mem_emb2 = _sc_walk_gather(entity_table, memory_graph, ent1.reshape(_B))
    logits = _phase_final_call(
        comb1, eemb2, W_proj, b_proj2, W_att, W_comb,
        b_comb2, W_score, mem_emb2.reshape(_B, _M, _E), entity_table)
    return logits


# speedup vs baseline: 1.6491x; 1.0160x over previous
import functools

import jax
import jax.numpy as jnp
from jax import lax
from jax.experimental import pallas as pl
from jax.experimental.pallas import tpu as pltpu
from jax.experimental.pallas import tpu_sc as plsc

_V = 100000
_E = 128
_H2 = 1024
_B = 64
_S = 50
_M = 32
_VT = 10240
_NV = (_V + _VT - 1) // _VT
_NW = 32


_TF_R1 = (13, 15, 26, 6)
_TF_R2 = (17, 29, 16, 24)
_TF_PARITY = 0x1BD11BDA


def _tf_py(k1, k2, x0, x1):
    msk = 0xFFFFFFFF
    rot = lambda x, r: ((x << r) | (x >> (32 - r))) & msk
    ks = [k1, k2, (k1 ^ k2 ^ _TF_PARITY) & msk]
    x0 = (x0 + ks[0]) & msk
    x1 = (x1 + ks[1]) & msk
    for j, rs in enumerate((_TF_R1, _TF_R2, _TF_R1, _TF_R2, _TF_R1)):
        for r in rs:
            x0 = (x0 + x1) & msk
            x1 = x0 ^ rot(x1, r)
        x0 = (x0 + ks[(j + 1) % 3]) & msk
        x1 = (x1 + ks[(j + 2) % 3] + j + 1) & msk
    return x0, x1


def _fold_in_py(i):
    return _tf_py(0, 42, 0, i)


def _i32c(x):
    x &= 0xFFFFFFFF
    return x - (1 << 32) if x >= (1 << 31) else x


def _tf_bits(k1, k2, p):
    c = lambda v: jnp.int32(_i32c(v))
    ks0, ks1 = k1, k2
    ks2 = k1 ^ k2 ^ _TF_PARITY

    def rot(x, r):
        return lax.shift_left(x, jnp.int32(r)) | lax.shift_right_logical(
            x, jnp.int32(32 - r))

    x0 = jnp.full(p.shape, c(ks0), jnp.int32)
    x1 = p + c(ks1)
    ks = (ks0, ks1, ks2)
    for j, rs in enumerate((_TF_R1, _TF_R2, _TF_R1, _TF_R2, _TF_R1)):
        for r in rs:
            x0 = x0 + x1
            x1 = x0 ^ rot(x1, r)
        x0 = x0 + c(ks[(j + 1) % 3])
        x1 = x1 + c(ks[(j + 2) % 3] + j + 1)
    return x0 ^ x1


_K0 = _fold_in_py(0)
_K1 = _fold_in_py(1)
_F32_TINY = 1.1754943508222875e-38


def _gumbel_tile(key_words, i):
    row = lax.broadcasted_iota(jnp.int32, (_B, _VT), 0)
    col = lax.broadcasted_iota(jnp.int32, (_B, _VT), 1) + i * _VT
    p = row * _V + col
    bits = _tf_bits(key_words[0], key_words[1], p)
    fb = lax.shift_right_logical(bits, jnp.int32(9)) | jnp.int32(0x3F800000)
    f = lax.bitcast_convert_type(fb, jnp.float32) - jnp.float32(1.0)
    u = jnp.maximum(f, jnp.float32(_F32_TINY))
    return -jnp.log(-jnp.log(u))


def _sc_mesh():
    return plsc.VectorSubcoreMesh(core_axis_name="c", subcore_axis_name="s")


def _sc_initial_gather(word_table, qidx_flat, entity_table, mem_flat):
    @functools.partial(
        pl.kernel,
        out_type=(
            jax.ShapeDtypeStruct((_NW * 104, _E), jnp.float32),
            jax.ShapeDtypeStruct((_B * _M, _E), jnp.float32),
        ),
        mesh=_sc_mesh(),
        scratch_types=[
            pltpu.VMEM((104,), jnp.int32),
            pltpu.VMEM((104, _E), jnp.float32),
            pltpu.VMEM((64,), jnp.int32),
            pltpu.VMEM((64, _E), jnp.float32),
            pltpu.SemaphoreType.DMA,
            pltpu.SemaphoreType.DMA,
        ],
    )
    def k(wt, qi, et, mf, qe_out, me_out, qi_v, qrows_v, mi_v, mrows_v, sem, sem2):
        wid = lax.axis_index("s") * 2 + lax.axis_index("c")
        pltpu.sync_copy(mf.at[pl.ds(wid * 64, 64)], mi_v)
        pltpu.async_copy(et.at[mi_v], mrows_v, sem2)
        pltpu.sync_copy(qi.at[pl.ds(wid * 128, 104)], qi_v)
        pltpu.async_copy(wt.at[qi_v], qrows_v, sem).wait()
        pltpu.sync_copy(qrows_v, qe_out.at[pl.ds(wid * 104, 104)])
        pltpu.make_async_copy(et.at[mi_v], mrows_v, sem2).wait()
        pltpu.sync_copy(mrows_v, me_out.at[pl.ds(wid * 64, 64)])

    return k(word_table, qidx_flat, entity_table, mem_flat)


def _sc_walk_gather(entity_table, memory_graph, entities):
    @functools.partial(
        pl.kernel,
        out_type=(
            jax.ShapeDtypeStruct((_B, _E), jnp.float32),
            jax.ShapeDtypeStruct((_B * _M, _E), jnp.float32),
        ),
        mesh=_sc_mesh(),
        scratch_types=[
            pltpu.VMEM((_B + 16,), jnp.int32),
            pltpu.VMEM((8, _M), jnp.int32),
            pltpu.VMEM((8, _M), jnp.int32),
            pltpu.VMEM((64,), jnp.int32),
            pltpu.VMEM((64, _E), jnp.float32),
            pltpu.VMEM((_B, _E), jnp.float32),
            pltpu.SemaphoreType.DMA,
            pltpu.SemaphoreType.DMA,
        ],
    )
    def k(et, mg, ents, eemb_out, memb_out, ents_v, blk0_v, blk1_v, idx_v,
          erows_v, eerows_v, sem, sem2):
        wid = lax.axis_index("s") * 2 + lax.axis_index("c")
        b0 = wid * 2
        pltpu.sync_copy(ents, ents_v.at[pl.ds(0, _B)])

        @pl.when(wid < 8)
        def _():
            pltpu.async_copy(et.at[ents_v.at[pl.ds(0, _B)]], eerows_v, sem2)

        e0 = ents_v[pl.ds(b0, 16)][0]
        e1 = ents_v[pl.ds(b0 + 1, 16)][0]
        ca = pltpu.async_copy(
            mg.at[pl.ds(pl.multiple_of((e0 // 8) * 8, 8), 8)], blk0_v, sem)
        cb = pltpu.async_copy(
            mg.at[pl.ds(pl.multiple_of((e1 // 8) * 8, 8), 8)], blk1_v, sem)
        ca.wait()
        cb.wait()
        for r, (blk, e) in enumerate(((blk0_v, e0), (blk1_v, e1))):
            for c in range(2):
                idx_v[pl.ds((r * 2 + c) * 16, 16)] = blk[e % 8, pl.ds(c * 16, 16)]
        pltpu.async_copy(et.at[idx_v], erows_v, sem).wait()
        pltpu.sync_copy(erows_v, memb_out.at[pl.ds(b0 * _M, 64)])

        @pl.when(wid < 8)
        def _():
            pltpu.make_async_copy(et.at[ents_v.at[pl.ds(0, _B)]], eerows_v,
                                  sem2).wait()
            pltpu.sync_copy(eerows_v.at[pl.ds(wid * 8, 8)],
                            eemb_out.at[pl.ds(wid * 8, 8)])

    return k(entity_table, memory_graph, entities)


def _front(aq, me, watt, wca, wcb, bc, ws):
    t = lax.dot_general(aq, watt, (((1,), (1,)), ((), ())),
                        preferred_element_type=jnp.float32)
    scores = jnp.sum(me * t[:, None, :], axis=2)
    scores = scores - jnp.max(scores, axis=1, keepdims=True)
    ex = jnp.exp(scores)
    w = ex / jnp.sum(ex, axis=1, keepdims=True)
    mv = jnp.sum(w[:, :, None] * me, axis=1)
    comb = jnp.tanh(
        jnp.dot(aq, wca, preferred_element_type=jnp.float32)
        + jnp.dot(mv, wcb, preferred_element_type=jnp.float32)
        + bc)
    s = jnp.dot(comb, ws, preferred_element_type=jnp.float32)
    return comb, s


def _select_tail(i, s_scr, et_ref, key_words, bv, bi, ent_out):
    lg = lax.dot_general(s_scr[...], et_ref[...], (((1,), (1,)), ((), ())),
                         preferred_element_type=jnp.float32) + _gumbel_tile(key_words, i)
    iota_v = lax.broadcasted_iota(jnp.int32, (_B, _VT), 1)
    lg = jnp.where(iota_v + i * _VT < _V, lg, -jnp.inf)
    mval = jnp.max(lg, axis=1, keepdims=True)
    idx = jnp.min(jnp.where(lg == mval, iota_v, _VT), axis=1, keepdims=True)
    gidx = idx + i * _VT
    upd = mval > bv[...]
    bi[...] = jnp.where(upd, gidx, bi[...])
    bv[...] = jnp.where(upd, mval, bv[...])

    @pl.when(i == _NV - 1)
    def _():
        ent_out[...] = bi[...]


def _phase0_call(q_emb3, len_f, W_enc, b_enc2, W_att, W_comb, b_comb2,
                 W_score, mem_emb3, entity_table, key_words):
    def body(q_ref, len_ref, wenc, benc, watt, wc, bc, ws, me, et_ref,
             comb_out, ent_out, s_scr, bv, bi):
        i = pl.program_id(0)

        @pl.when(i == 0)
        def _():
            iota_s = lax.broadcasted_iota(jnp.int32, (_B, _S), 1).astype(jnp.float32)
            mask = (iota_s < len_ref[...]).astype(jnp.float32)
            qp = jnp.sum(q_ref[...] * mask[:, :, None], axis=1) / len_ref[...]
            aq = jnp.tanh(jnp.dot(qp, wenc[...],
                                  preferred_element_type=jnp.float32) + benc[...])
            comb, s = _front(aq, me[...], watt[...], wc[0:_H2, :],
                             wc[_H2:, :], bc[...], ws[...])
            comb_out[...] = comb
            s_scr[...] = s
            bv[...] = jnp.full((_B, 1), -jnp.inf, jnp.float32)
            bi[...] = jnp.zeros((_B, 1), jnp.int32)

        _select_tail(i, s_scr, et_ref, key_words, bv, bi, ent_out)

    full = lambda shape: pl.BlockSpec(shape, lambda i: (0,) * len(shape))
    return pl.pallas_call(
        body,
        grid=(_NV,),
        in_specs=[
            full((_B, _S, _E)),
            full((_B, 1)),
            full((_E, _H2)),
            full((1, _H2)),
            full((_E, _H2)),
            full((_H2 + _E, _H2)),
            full((1, _H2)),
            full((_H2, _E)),
            full((_B, _M, _E)),
            pl.BlockSpec((_VT, _E), lambda i: (i, 0)),
        ],
        out_specs=[full((_B, _H2)), full((_B, 1))],
        out_shape=[
            jax.ShapeDtypeStruct((_B, _H2), jnp.float32),
            jax.ShapeDtypeStruct((_B, 1), jnp.int32),
        ],
        scratch_shapes=[
            pltpu.VMEM((_B, _E), jnp.float32),
            pltpu.VMEM((_B, 1), jnp.float32),
            pltpu.VMEM((_B, 1), jnp.int32),
        ],
        compiler_params=pltpu.CompilerParams(
            dimension_semantics=("arbitrary",)),
    )(q_emb3, len_f, W_enc, b_enc2, W_att, W_comb, b_comb2,
      W_score, mem_emb3, entity_table)


def _phase_walk_call(comb_prev, eemb, W_proj, b_proj2, W_att,
                     W_comb, b_comb2, W_score, mem_emb3,
                     entity_table, key_words):
    def body(cp, ee, wp, bp, watt, wc, bc, ws, me, et_ref,
             comb_out, ent_out, s_scr, bv, bi):
        i = pl.program_id(0)

        @pl.when(i == 0)
        def _():
            aq = jnp.tanh(
                jnp.dot(cp[...], wp[0:_H2, :], preferred_element_type=jnp.float32)
                + jnp.dot(ee[...], wp[_H2:, :], preferred_element_type=jnp.float32)
                + bp[...])
            comb, s = _front(aq, me[...], watt[...], wc[0:_H2, :],
                             wc[_H2:, :], bc[...], ws[...])
            comb_out[...] = comb
            s_scr[...] = s
            bv[...] = jnp.full((_B, 1), -jnp.inf, jnp.float32)
            bi[...] = jnp.zeros((_B, 1), jnp.int32)

        _select_tail(i, s_scr, et_ref, key_words, bv, bi, ent_out)

    full = lambda shape: pl.BlockSpec(shape, lambda i: (0,) * len(shape))
    return pl.pallas_call(
        body,
        grid=(_NV,),
        in_specs=[
            full((_B, _H2)),
            full((_B, _E)),
            full((_H2 + _E, _H2)),
            full((1, _H2)),
            full((_E, _H2)),
            full((_H2 + _E, _H2)),
            full((1, _H2)),
            full((_H2, _E)),
            full((_B, _M, _E)),
            pl.BlockSpec((_VT, _E), lambda i: (i, 0)),
        ],
        out_specs=[full((_B, _H2)), full((_B, 1))],
        out_shape=[
            jax.ShapeDtypeStruct((_B, _H2), jnp.float32),
            jax.ShapeDtypeStruct((_B, 1), jnp.int32),
        ],
        scratch_shapes=[
            pltpu.VMEM((_B, _E), jnp.float32),
            pltpu.VMEM((_B, 1), jnp.float32),
            pltpu.VMEM((_B, 1), jnp.int32),
        ],
        compiler_params=pltpu.CompilerParams(
            dimension_semantics=("arbitrary",)),
    )(comb_prev, eemb, W_proj, b_proj2, W_att, W_comb,
      b_comb2, W_score, mem_emb3, entity_table)


def _phase_final_call(comb_prev, eemb, W_proj, b_proj2, W_att,
                      W_comb, b_comb2, W_score, mem_emb3,
                      entity_table):
    def body(cp, ee, wp, bp, watt, wc, bc, ws, me, et_ref,
             out_ref, s_scr):
        i = pl.program_id(0)

        @pl.when(i == 0)
        def _():
            aq = jnp.tanh(
                jnp.dot(cp[...], wp[0:_H2, :], preferred_element_type=jnp.float32)
                + jnp.dot(ee[...], wp[_H2:, :], preferred_element_type=jnp.float32)
                + bp[...])
            _, s = _front(aq, me[...], watt[...], wc[0:_H2, :],
                          wc[_H2:, :], bc[...], ws[...])
            s_scr[...] = s

        out_ref[...] = lax.dot_general(
            s_scr[...], et_ref[...], (((1,), (1,)), ((), ())),
            preferred_element_type=jnp.float32)

    full = lambda shape: pl.BlockSpec(shape, lambda i: (0,) * len(shape))
    return pl.pallas_call(
        body,
        grid=(_NV,),
        in_specs=[
            full((_B, _H2)),
            full((_B, _E)),
            full((_H2 + _E, _H2)),
            full((1, _H2)),
            full((_E, _H2)),
            full((_H2 + _E, _H2)),
            full((1, _H2)),
            full((_H2, _E)),
            full((_B, _M, _E)),
            pl.BlockSpec((_VT, _E), lambda i: (i, 0)),
        ],
        out_specs=pl.BlockSpec((_B, _VT), lambda i: (0, i)),
        out_shape=jax.ShapeDtypeStruct((_B, _V), jnp.float32),
        scratch_shapes=[pltpu.VMEM((_B, _E), jnp.float32)],
        compiler_params=pltpu.CompilerParams(
            dimension_semantics=("arbitrary",)),
    )(comb_prev, eemb, W_proj, b_proj2, W_att, W_comb,
      b_comb2, W_score, mem_emb3, entity_table)


def kernel(memory, question, question_lengths, memory_graph, entity_table,
           word_table, W_enc, b_enc, W_att, W_comb, b_comb, W_score, W_proj,
           b_proj):
    memory = memory.astype(jnp.int32)
    question = question.astype(jnp.int32)
    memory_graph = memory_graph.astype(jnp.int32)

    len_f = jnp.maximum(question_lengths, 1).astype(jnp.float32).reshape(_B, 1)
    b_enc2 = b_enc.reshape(1, _H2)
    b_comb2 = b_comb.reshape(1, _H2)
    b_proj2 = b_proj.reshape(1, _H2)

    qflat = jnp.pad(question.reshape(_B * _S), (0, _NW * 104 - _B * _S))
    qidx_flat = jnp.pad(qflat.reshape(_NW, 104), ((0, 0), (0, 24))).reshape(_NW * 128)
    mem_flat = memory.reshape(_B * _M)

    q_emb, mem_emb0 = _sc_initial_gather(word_table, qidx_flat, entity_table,
                                         mem_flat)
    comb0, ent0 = _phase0_call(
        q_emb[:_B * _S].reshape(_B, _S, _E), len_f, W_enc, b_enc2, W_att,
        W_comb, b_comb2, W_score, mem_emb0.reshape(_B, _M, _E),
        entity_table, _K0)

    eemb1, mem_emb1 = _sc_walk_gather(entity_table, memory_graph, ent0.reshape(_B))
    comb1, ent1 = _phase_walk_call(
        comb0, eemb1, W_proj, b_proj2, W_att, W_comb,
        b_comb2, W_score, mem_emb1.reshape(_B, _M, _E), entity_table, _K1)

    eemb2, mem_emb2 = _sc_walk_gather(entity_table, memory_graph, ent1.reshape(_B))
    logits = _phase_final_call(
        comb1, eemb2, W_proj, b_proj2, W_att, W_comb,
        b_comb2, W_score, mem_emb2.reshape(_B, _M, _E), entity_table)
    return logits
